# baseline (speedup 1.0000x reference)
import jax
import jax.numpy as jnp
from jax.experimental import pallas as pl


def _leaky(x):
    return jnp.where(x >= 0, x, 0.01 * x)


def _apply_net(layers, x):
    n = len(layers)
    for i, (W, b) in enumerate(layers):
        x = x @ W + b
        if i < n - 1:
            x = _leaky(x)
    return x


def _wap(p, x, index, weights, num_segments):
    gate = _apply_net(p["gate"], x)
    gmax = jax.ops.segment_max(gate, index, num_segments=num_segments)
    gate = gate - gmax[index]
    gate = (weights ** p["pow"]) * jnp.exp(gate)
    denom = jax.ops.segment_sum(gate, index, num_segments=num_segments)
    gate = gate / (denom[index] + 1e-10)
    msg = _apply_net(p["msg"], x)
    return jax.ops.segment_sum(gate * msg, index, num_segments=num_segments)


def _mean3_kernel(a_ref, b_ref, c_ref, o_ref):
    o_ref[...] = (a_ref[...] + b_ref[...] + c_ref[...]) * (1.0 / 3.0)


def kernel(elem_weights, elem_fea, params, self_fea_idx, nbr_fea_idx, cry_elem_idx):
    W, b = params["emb"]
    fea = elem_fea @ W + b
    fea = jnp.concatenate([fea, elem_weights], axis=1)
    N = fea.shape[0]
    C = 1000
    for heads in params["graphs"]:
        nbr_w = elem_weights[nbr_fea_idx]
        self_f = fea[self_fea_idx]
        nbr_f = fea[nbr_fea_idx]
        pair = jnp.concatenate([self_f, nbr_f], axis=1)
        hf = [_wap(p, pair, self_fea_idx, nbr_w, N) for p in heads]
        fea = jnp.mean(jnp.stack(hf), axis=0) + fea
    hf = [_wap(p, fea, cry_elem_idx, elem_weights, C) for p in params["cry"]]
    return pl.pallas_call(
        _mean3_kernel,
        out_shape=jax.ShapeDtypeStruct(hf[0].shape, hf[0].dtype),
    )(hf[0], hf[1], hf[2])


# speedup vs baseline: 5.0967x; 5.0967x over previous
import functools

import jax
import jax.numpy as jnp
from jax import lax
from jax.experimental import pallas as pl
from jax.experimental.pallas import tpu as pltpu
from jax.experimental.pallas import tpu_sc as plsc

N = 10000
M = 320000
C = 1000
F = 64
NB = 1000
GN = N // NB
CB = 128
GC = (C + CB - 1) // CB
ACCW = 208
TSW = 128
TNW = 144
PACC = 80
NW = 32
EPW = M // NW
EB = 80
NBATCH = EPW // EB


def _leaky(x):
    return jnp.where(x >= 0, x, 0.01 * x)


def _emb_body(ef_ref, w_ref, b_ref, ew_ref, fea_ref, logw_ref):
    y = jnp.dot(ef_ref[...], w_ref[...], preferred_element_type=jnp.float32)
    y = y + b_ref[...]
    ew = ew_ref[...]
    mask = lax.broadcasted_iota(jnp.int32, (NB, F), 1) == (F - 1)
    fea_ref[...] = y + jnp.where(mask, ew, 0.0)
    logw_ref[...] = jnp.log(ew)


def _embed(elem_fea, elem_weights, W64, b64):
    return pl.pallas_call(
        _emb_body,
        grid=(GN,),
        in_specs=[
            pl.BlockSpec((NB, 200), lambda i: (i, 0)),
            pl.BlockSpec((200, F), lambda i: (0, 0)),
            pl.BlockSpec((1, F), lambda i: (0, 0)),
            pl.BlockSpec((NB, 1), lambda i: (i, 0)),
        ],
        out_specs=[
            pl.BlockSpec((NB, F), lambda i: (i, 0)),
            pl.BlockSpec((NB, 1), lambda i: (i, 0)),
        ],
        out_shape=[
            jax.ShapeDtypeStruct((N, F), jnp.float32),
            jax.ShapeDtypeStruct((N, 1), jnp.float32),
        ],
    )(elem_fea, W64, b64, elem_weights)


def _tables_body(fea_ref, logw_ref, ws_ref, bs_ref, wn_ref, pow_ref,
                 ts_ref, tn_ref):
    fea = fea_ref[...]
    ts_ref[...] = (
        jnp.dot(fea, ws_ref[0], preferred_element_type=jnp.float32)
        + bs_ref[0])
    tn_ref[:, :TSW] = jnp.dot(fea, wn_ref[0],
                              preferred_element_type=jnp.float32)
    tn_ref[:, TSW:TNW] = logw_ref[...] * pow_ref[0, 0:1, 0:16]


def _tables(fea, logw, Ws3, bs3, Wn3, pow3):
    return pl.pallas_call(
        _tables_body,
        grid=(3, GN),
        in_specs=[
            pl.BlockSpec((NB, F), lambda h, i: (i, 0)),
            pl.BlockSpec((NB, 1), lambda h, i: (i, 0)),
            pl.BlockSpec((1, F, TSW), lambda h, i: (h, 0, 0)),
            pl.BlockSpec((1, 1, TSW), lambda h, i: (h, 0, 0)),
            pl.BlockSpec((1, F, TSW), lambda h, i: (h, 0, 0)),
            pl.BlockSpec((1, 1, 128), lambda h, i: (h, 0, 0)),
        ],
        out_specs=[
            pl.BlockSpec((NB, TSW), lambda h, i: (h * GN + i, 0)),
            pl.BlockSpec((NB, TNW), lambda h, i: (h * GN + i, 0)),
        ],
        out_shape=[
            jax.ShapeDtypeStruct((3 * N, TSW), jnp.float32),
            jax.ShapeDtypeStruct((3 * N, TNW), jnp.float32),
        ],
    )(fea, logw, Ws3, bs3, Wn3, pow3)


def _gbound_body(ts_ref, tn_ref, w2g_ref, g_ref, acc_ref):
    h = pl.program_id(0)
    i = pl.program_id(1)
    a = ts_ref[:, :64]
    b = tn_ref[:, :64]
    alo = jnp.min(a, axis=0, keepdims=True)
    ahi = jnp.max(a, axis=0, keepdims=True)
    blo = jnp.min(b, axis=0, keepdims=True)
    bhi = jnp.max(b, axis=0, keepdims=True)
    cur = jnp.concatenate([alo, ahi, blo, bhi], axis=0)

    @pl.when(i == 0)
    def _():
        acc_ref[0:4, :] = cur

    @pl.when(i > 0)
    def _():
        old = acc_ref[0:4, :]
        lo = jnp.minimum(old, cur)
        hi = jnp.maximum(old, cur)
        sel = lax.broadcasted_iota(jnp.int32, (4, 64), 0) % 2
        acc_ref[0:4, :] = jnp.where(sel == 0, lo, hi)

    @pl.when(jnp.logical_and(h == 0, i == 0))
    def _():
        g_ref[...] = jnp.zeros((1, 128), jnp.float32)

    @pl.when(i == GN - 1)
    def _():
        lane = lax.broadcasted_iota(jnp.int32, (1, 128), 1)
        w = w2g_ref[0]
        t = jnp.maximum(w * _leaky(acc_ref[1:2, :] + acc_ref[3:4, :]),
                        w * _leaky(acc_ref[0:1, :] + acc_ref[2:3, :]))
        g_ref[...] = g_ref[...] + jnp.sum(t) * jnp.where(lane == h, 1.0, 0.0)


def _gbound(TS, TN, w2g3):
    return pl.pallas_call(
        _gbound_body,
        grid=(3, GN),
        in_specs=[
            pl.BlockSpec((NB, TSW), lambda h, i: (h * GN + i, 0)),
            pl.BlockSpec((NB, TNW), lambda h, i: (h * GN + i, 0)),
            pl.BlockSpec((1, 1, 64), lambda h, i: (h, 0, 0)),
        ],
        out_specs=pl.BlockSpec((1, 128), lambda h, i: (0, 0)),
        out_shape=jax.ShapeDtypeStruct((1, 128), jnp.float32),
        scratch_shapes=[pltpu.VMEM((8, 64), jnp.float32)],
    )(TS, TN, w2g3)


def _make_edge_body(hoff):
    def _edge_body(ts_hbm, tn_hbm, sidx_hbm, nidx_hbm, w2g_hbm, g_hbm,
                   out_hbm, idx_s, idx_gs, idx_n, rows_s, rows_n, stage,
                   wv, gv, zbuf, acc, sem1, sem2):
        c = lax.axis_index("c")
        s = lax.axis_index("s")
        wid = s * 2 + c
        lane0 = lax.iota(jnp.int32, 16) == 0

        def _zrow(r, _):
            for k in range(PACC // 16):
                zbuf[r, pl.ds(k * 16, 16)] = jnp.zeros((16,), jnp.float32)
            return 0

        lax.fori_loop(0, 40, _zrow, 0)
        nchunks = N // 40
        for j in range(16):
            chunk = s + 16 * j

            @pl.when(chunk < nchunks)
            def _():
                pltpu.sync_copy(zbuf, acc.at[pl.ds(chunk * 40, 40)])

        pltpu.sync_copy(w2g_hbm, wv)
        pltpu.sync_copy(g_hbm, gv)
        gsh = gv[...][0]
        plsc.subcore_barrier()

        def _batch(bi, _):
            base = wid * EPW + bi * EB
            pltpu.sync_copy(sidx_hbm.at[pl.ds(base, EB)], idx_s)
            pltpu.sync_copy(nidx_hbm.at[pl.ds(base, EB)], idx_n)
            for k in range(EB // 16):
                idx_gs[pl.ds(k * 16, 16)] = idx_s[pl.ds(k * 16, 16)] + hoff
                idx_n[pl.ds(k * 16, 16)] = idx_n[pl.ds(k * 16, 16)] + hoff
            cp1 = pltpu.async_copy(ts_hbm.at[idx_gs], rows_s, sem1)
            cp2 = pltpu.async_copy(tn_hbm.at[idx_n], rows_n, sem2)
            cp1.wait()
            cp2.wait()

            def _edge(e, _):
                lwv = rows_n[e, pl.ds(TSW, 16)]
                pacc = jnp.zeros((16,), jnp.float32)
                for k in range(4):
                    off = k * 16
                    x = rows_s[e, pl.ds(off, 16)] + rows_n[e, pl.ds(off, 16)]
                    hg = jnp.where(x >= 0, x, 0.01 * x)
                    pacc = pacc + hg * wv[pl.ds(off, 16)]
                targ = jnp.sum(pacc) + lwv[0] - gsh
                tv = jnp.exp(jnp.full((16,), targ, jnp.float32))
                for k in range(4):
                    off = 64 + k * 16
                    x = rows_s[e, pl.ds(off, 16)] + rows_n[e, pl.ds(off, 16)]
                    hm = jnp.where(x >= 0, x, 0.01 * x)
                    stage[e, pl.ds(k * 16, 16)] = tv * hm
                stage[e, pl.ds(64, 16)] = jnp.where(lane0, tv, 0.0)
                return 0

            lax.fori_loop(0, EB, _edge, 0)
            pltpu.sync_copy(stage, acc.at[idx_s], add=True)
            return 0

        lax.fori_loop(0, NBATCH, _batch, 0)
        plsc.subcore_barrier()

        for j in range(16):
            chunk = s + 16 * j

            @pl.when(chunk < nchunks)
            def _():
                pltpu.sync_copy(acc.at[pl.ds(chunk * 40, 40)], zbuf)
                pltpu.sync_copy(zbuf, out_hbm.at[c, pl.ds(chunk * 40, 40)])

    return _edge_body


def _edge_pass(TS, TN, sidx, nidx, w2g64, g16, hoff):
    mesh = plsc.VectorSubcoreMesh(core_axis_name="c", subcore_axis_name="s")
    fn = functools.partial(
        pl.kernel,
        mesh=mesh,
        compiler_params=pltpu.CompilerParams(
            use_tc_tiling_on_sc=False, needs_layout_passes=False),
        out_type=jax.ShapeDtypeStruct((2, N, PACC), jnp.float32),
        scratch_types=[
            pltpu.VMEM((EB,), jnp.int32),
            pltpu.VMEM((EB,), jnp.int32),
            pltpu.VMEM((EB,), jnp.int32),
            pltpu.VMEM((EB, TSW), jnp.float32),
            pltpu.VMEM((EB, TNW), jnp.float32),
            pltpu.VMEM((EB, PACC), jnp.float32),
            pltpu.VMEM((F,), jnp.float32),
            pltpu.VMEM((16,), jnp.float32),
            pltpu.VMEM((40, PACC), jnp.float32),
            pltpu.VMEM_SHARED((N, PACC), jnp.float32),
            pltpu.SemaphoreType.DMA,
            pltpu.SemaphoreType.DMA,
        ],
    )(_make_edge_body(hoff))
    return fn(TS, TN, sidx, nidx, w2g64, g16)


def _combine_body(p0_ref, p1_ref, p2_ref, fea_ref, w2bd_ref, b2_ref,
                  out_ref):
    ps = [p0_ref[0] + p0_ref[1], p1_ref[0] + p1_ref[1], p2_ref[0] + p2_ref[1]]
    numer = jnp.concatenate([p[:, :F] for p in ps], axis=1)
    msgs = jnp.dot(numer, w2bd_ref[...], preferred_element_type=jnp.float32)
    tot = jnp.zeros((NB, F), jnp.float32)
    for h in range(3):
        denom = ps[h][:, F:F + 1]
        o = (msgs[:, h * F:(h + 1) * F]
             + denom * b2_ref[0:1, h * F:(h + 1) * F])
        tot = tot + o / (denom + 1e-30)
    out_ref[...] = fea_ref[...] + tot * (1.0 / 3.0)


def _combine(P0, P1, P2, fea, W2bd, b2cat):
    pspec = pl.BlockSpec((2, NB, PACC), lambda i: (0, i, 0))
    return pl.pallas_call(
        _combine_body,
        grid=(GN,),
        in_specs=[
            pspec, pspec, pspec,
            pl.BlockSpec((NB, F), lambda i: (i, 0)),
            pl.BlockSpec((192, 192), lambda i: (0, 0)),
            pl.BlockSpec((1, 192), lambda i: (0, 0)),
        ],
        out_specs=pl.BlockSpec((NB, F), lambda i: (i, 0)),
        out_shape=jax.ShapeDtypeStruct((N, F), jnp.float32),
    )(P0, P1, P2, fea, W2bd, b2cat)


def _crynode_body(fea_ref, logw_ref, w1g_ref, b1g_ref, w2gp_ref, b2gp_ref,
                  powp_ref, w1m_ref, b1m_ref, x_ref):
    fea = fea_ref[...]
    hg = _leaky(jnp.dot(fea, w1g_ref[...], preferred_element_type=jnp.float32)
                + b1g_ref[...])
    logit = (jnp.dot(hg, w2gp_ref[...], preferred_element_type=jnp.float32)
             + b2gp_ref[...] + logw_ref[...] * powp_ref[...])
    hm = _leaky(jnp.dot(fea, w1m_ref[...], preferred_element_type=jnp.float32)
                + b1m_ref[...])
    x_ref[:, :192] = hm
    x_ref[:, 192:ACCW] = logit[:, 0:16]


def _crynode(fea, logw, W1g_cat, b1g_cat, W2g_pad, b2g_pad, pow_pad,
             W1m_cat, b1m_cat):
    return pl.pallas_call(
        _crynode_body,
        grid=(GN,),
        in_specs=[
            pl.BlockSpec((NB, F), lambda i: (i, 0)),
            pl.BlockSpec((NB, 1), lambda i: (i, 0)),
            pl.BlockSpec((F, 192), lambda i: (0, 0)),
            pl.BlockSpec((1, 192), lambda i: (0, 0)),
            pl.BlockSpec((192, 128), lambda i: (0, 0)),
            pl.BlockSpec((1, 128), lambda i: (0, 0)),
            pl.BlockSpec((1, 128), lambda i: (0, 0)),
            pl.BlockSpec((F, 192), lambda i: (0, 0)),
            pl.BlockSpec((1, 192), lambda i: (0, 0)),
        ],
        out_specs=pl.BlockSpec((NB, ACCW), lambda i: (i, 0)),
        out_shape=jax.ShapeDtypeStruct((N, ACCW), jnp.float32),
    )(fea, logw, W1g_cat, b1g_cat, W2g_pad, b2g_pad, pow_pad,
      W1m_cat, b1m_cat)


def _crymax_body(x_ref, cry_ref, gm_ref):
    ci = pl.program_id(0)
    ni = pl.program_id(1)
    cids = ci * CB + lax.broadcasted_iota(jnp.int32, (NB, CB), 1)
    mask = cry_ref[...] == cids
    cur = jnp.full((8, CB), -1e30, jnp.float32)
    row = lax.broadcasted_iota(jnp.int32, (8, CB), 0)
    for h in range(3):
        lh = x_ref[:, 192 + h:193 + h]
        mh = jnp.max(jnp.where(mask, lh, -1e30), axis=0, keepdims=True)
        cur = jnp.where(row == h, mh, cur)

    @pl.when(ni == 0)
    def _():
        gm_ref[...] = cur

    @pl.when(ni > 0)
    def _():
        gm_ref[...] = jnp.maximum(gm_ref[...], cur)


def _crymax(X, cry2d):
    return pl.pallas_call(
        _crymax_body,
        grid=(GC, GN),
        in_specs=[
            pl.BlockSpec((NB, ACCW), lambda ci, ni: (ni, 0)),
            pl.BlockSpec((NB, 1), lambda ci, ni: (ni, 0)),
        ],
        out_specs=pl.BlockSpec((8, CB), lambda ci, ni: (0, ci)),
        out_shape=jax.ShapeDtypeStruct((8, GC * CB), jnp.float32),
    )(X, cry2d)


def _crysum_body(x_ref, cry_ref, gm_ref, num_ref, d_ref):
    ci = pl.program_id(0)
    ni = pl.program_id(1)
    cids = ci * CB + lax.broadcasted_iota(jnp.int32, (NB, CB), 1)
    mask = cry_ref[...] == cids
    nums = []
    d128 = jnp.zeros((CB, CB), jnp.float32)
    lane = lax.broadcasted_iota(jnp.int32, (CB, CB), 1)
    for h in range(3):
        lh = x_ref[:, 192 + h:193 + h]
        gmh = gm_ref[h:h + 1, :]
        t = jnp.where(mask, jnp.exp(lh - gmh), 0.0)
        hm = x_ref[:, h * 64:(h + 1) * 64]
        nums.append(lax.dot_general(t, hm, (((0,), (0,)), ((), ())),
                                    preferred_element_type=jnp.float32))
        sh = jnp.sum(t, axis=0)
        d128 = d128 + sh[:, None] * jnp.where(lane == h, 1.0, 0.0)
    cur = jnp.concatenate(nums, axis=1)

    @pl.when(ni == 0)
    def _():
        num_ref[...] = cur
        d_ref[...] = d128

    @pl.when(ni > 0)
    def _():
        num_ref[...] = num_ref[...] + cur
        d_ref[...] = d_ref[...] + d128


def _crysum(X, cry2d, GM):
    return pl.pallas_call(
        _crysum_body,
        grid=(GC, GN),
        in_specs=[
            pl.BlockSpec((NB, ACCW), lambda ci, ni: (ni, 0)),
            pl.BlockSpec((NB, 1), lambda ci, ni: (ni, 0)),
            pl.BlockSpec((8, CB), lambda ci, ni: (0, ci)),
        ],
        out_specs=[
            pl.BlockSpec((CB, 192), lambda ci, ni: (ci, 0)),
            pl.BlockSpec((CB, CB), lambda ci, ni: (ci, 0)),
        ],
        out_shape=[
            jax.ShapeDtypeStruct((GC * CB, 192), jnp.float32),
            jax.ShapeDtypeStruct((GC * CB, CB), jnp.float32),
        ],
    )(X, cry2d, GM)


def _cryfin_body(num_ref, d_ref, w2bd_ref, b2_ref, out_ref):
    msgs = jnp.dot(num_ref[...], w2bd_ref[...],
                   preferred_element_type=jnp.float32)
    tot = jnp.zeros((C, F), jnp.float32)
    for h in range(3):
        denom = d_ref[:, h:h + 1]
        o = (msgs[:, h * 64:(h + 1) * 64]
             + denom * b2_ref[0:1, h * 64:(h + 1) * 64])
        tot = tot + o / (denom + 1e-30)
    out_ref[...] = tot * (1.0 / 3.0)


def _cryfin(NUM, D, W2bd, b2cat):
    return pl.pallas_call(
        _cryfin_body,
        grid=(1,),
        in_specs=[
            pl.BlockSpec((C, 192), lambda i: (0, 0)),
            pl.BlockSpec((C, CB), lambda i: (0, 0)),
            pl.BlockSpec((192, 192), lambda i: (0, 0)),
            pl.BlockSpec((1, 192), lambda i: (0, 0)),
        ],
        out_specs=pl.BlockSpec((C, F), lambda i: (0, 0)),
        out_shape=jax.ShapeDtypeStruct((C, F), jnp.float32),
    )(NUM[:C], D[:C], W2bd, b2cat)


def kernel(elem_weights, elem_fea, params, self_fea_idx, nbr_fea_idx,
           cry_elem_idx):
    Wemb, bemb = params["emb"]
    W64 = jnp.pad(Wemb, ((0, 0), (0, 1)))
    b64 = jnp.pad(bemb, (0, 1)).reshape(1, F)
    fea, logw = _embed(elem_fea, elem_weights, W64, b64)

    sidx = self_fea_idx.astype(jnp.int32)
    nidx = nbr_fea_idx.astype(jnp.int32)

    for heads in params["graphs"]:
        Ws_l, bs_l, Wn_l, pow_l, w2g_l, W2m, b2m = [], [], [], [], [], [], []
        for p in heads:
            (W1g, b1g), (w2g_, _) = p["gate"]
            (W1m, b1m), (W2m_, b2m_) = p["msg"]
            Ws_l.append(jnp.concatenate([W1g[:F], W1m[:F]], axis=1))
            bs_l.append(jnp.concatenate([b1g, b1m]).reshape(1, TSW))
            Wn_l.append(jnp.concatenate([W1g[F:], W1m[F:]], axis=1))
            pw = jnp.zeros((1, 128), jnp.float32).at[0, 0].set(p["pow"][0])
            pow_l.append(pw)
            w2g_l.append(w2g_[:, 0].reshape(1, F))
            W2m.append(W2m_)
            b2m.append(b2m_)
        Ws3 = jnp.stack(Ws_l)
        bs3 = jnp.stack(bs_l)
        Wn3 = jnp.stack(Wn_l)
        pow3 = jnp.stack(pow_l)
        w2g3 = jnp.stack(w2g_l)
        TS, TN = _tables(fea, logw, Ws3, bs3, Wn3, pow3)
        G = _gbound(TS, TN, w2g3)

        Ps = []
        for h in range(3):
            g16 = jnp.broadcast_to(G[0, h], (16,))
            Ps.append(_edge_pass(TS, TN, sidx, nidx, w2g3[h, 0], g16,
                                 h * N))
        W2bd = jax.scipy.linalg.block_diag(*W2m)
        b2cat = jnp.concatenate(b2m).reshape(1, 192)
        fea = _combine(Ps[0], Ps[1], Ps[2], fea, W2bd, b2cat)

    W1g_cat = jnp.concatenate([p["gate"][0][0] for p in params["cry"]],
                              axis=1)
    b1g_cat = jnp.concatenate([p["gate"][0][1] for p in params["cry"]]
                              ).reshape(1, 192)
    W2g_bd = jax.scipy.linalg.block_diag(
        *[p["gate"][1][0] for p in params["cry"]])
    W2g_pad = jnp.pad(W2g_bd, ((0, 0), (0, 125)))
    b2g_pad = jnp.zeros((1, 128), jnp.float32).at[0, :3].set(
        jnp.stack([p["gate"][1][1][0] for p in params["cry"]]))
    pow_pad = jnp.zeros((1, 128), jnp.float32).at[0, :3].set(
        jnp.stack([p["pow"][0] for p in params["cry"]]))
    W1m_cat = jnp.concatenate([p["msg"][0][0] for p in params["cry"]],
                              axis=1)
    b1m_cat = jnp.concatenate([p["msg"][0][1] for p in params["cry"]]
                              ).reshape(1, 192)
    X = _crynode(fea, logw, W1g_cat, b1g_cat, W2g_pad, b2g_pad, pow_pad,
                 W1m_cat, b1m_cat)
    cry2d = cry_elem_idx.astype(jnp.int32).reshape(N, 1)
    GM = _crymax(X, cry2d)
    NUM, D = _crysum(X, cry2d, GM)
    W2bd_c = jax.scipy.linalg.block_diag(
        *[p["msg"][1][0] for p in params["cry"]])
    b2cat_c = jnp.concatenate([p["msg"][1][1] for p in params["cry"]]
                              ).reshape(1, 192)
    return _cryfin(NUM, D, W2bd_c, b2cat_c)


# speedup vs baseline: 15.5985x; 3.0605x over previous
import functools

import jax
import jax.numpy as jnp
from jax import lax
from jax.experimental import pallas as pl
from jax.experimental.pallas import tpu as pltpu
from jax.experimental.pallas import tpu_sc as plsc

N = 10000
M = 320000
C = 1000
F = 64
NB = 1000
GN = N // NB
CB = 128
GC = (C + CB - 1) // CB
ACCW = 208
TSW = 128
TNW = 144
PACC = 80
NW = 32
EPW = M // NW
EB = 80
NBATCH = EPW // EB


def _leaky(x):
    return jnp.where(x >= 0, x, 0.01 * x)


def _emb_body(ef_ref, w_ref, b_ref, ew_ref, fea_ref, logw_ref):
    y = jnp.dot(ef_ref[...], w_ref[...], preferred_element_type=jnp.float32)
    y = y + b_ref[...]
    ew = ew_ref[...]
    mask = lax.broadcasted_iota(jnp.int32, (NB, F), 1) == (F - 1)
    fea_ref[...] = y + jnp.where(mask, ew, 0.0)
    logw_ref[...] = jnp.log(ew)


def _embed(elem_fea, elem_weights, W64, b64):
    return pl.pallas_call(
        _emb_body,
        grid=(GN,),
        in_specs=[
            pl.BlockSpec((NB, 200), lambda i: (i, 0)),
            pl.BlockSpec((200, F), lambda i: (0, 0)),
            pl.BlockSpec((1, F), lambda i: (0, 0)),
            pl.BlockSpec((NB, 1), lambda i: (i, 0)),
        ],
        out_specs=[
            pl.BlockSpec((NB, F), lambda i: (i, 0)),
            pl.BlockSpec((NB, 1), lambda i: (i, 0)),
        ],
        out_shape=[
            jax.ShapeDtypeStruct((N, F), jnp.float32),
            jax.ShapeDtypeStruct((N, 1), jnp.float32),
        ],
    )(elem_fea, W64, b64, elem_weights)


def _tables_body(fea_ref, logw_ref, ws_ref, bs_ref, wn_ref, pow_ref,
                 ts_ref, tn_ref):
    fea = fea_ref[...]
    ts_ref[...] = (
        jnp.dot(fea, ws_ref[0], preferred_element_type=jnp.float32)
        + bs_ref[0])
    tn_ref[:, :TSW] = jnp.dot(fea, wn_ref[0],
                              preferred_element_type=jnp.float32)
    tn_ref[:, TSW:TNW] = logw_ref[...] * pow_ref[0, 0:1, 0:16]


def _tables(fea, logw, Ws3, bs3, Wn3, pow3):
    return pl.pallas_call(
        _tables_body,
        grid=(3, GN),
        in_specs=[
            pl.BlockSpec((NB, F), lambda h, i: (i, 0)),
            pl.BlockSpec((NB, 1), lambda h, i: (i, 0)),
            pl.BlockSpec((1, F, TSW), lambda h, i: (h, 0, 0)),
            pl.BlockSpec((1, 1, TSW), lambda h, i: (h, 0, 0)),
            pl.BlockSpec((1, F, TSW), lambda h, i: (h, 0, 0)),
            pl.BlockSpec((1, 1, 128), lambda h, i: (h, 0, 0)),
        ],
        out_specs=[
            pl.BlockSpec((NB, TSW), lambda h, i: (h * GN + i, 0)),
            pl.BlockSpec((NB, TNW), lambda h, i: (h * GN + i, 0)),
        ],
        out_shape=[
            jax.ShapeDtypeStruct((3 * N, TSW), jnp.float32),
            jax.ShapeDtypeStruct((3 * N, TNW), jnp.float32),
        ],
    )(fea, logw, Ws3, bs3, Wn3, pow3)


def _gbound_body(ts_ref, tn_ref, w2g_ref, g_ref, acc_ref):
    h = pl.program_id(0)
    i = pl.program_id(1)
    a = ts_ref[:, :64]
    b = tn_ref[:, :64]
    alo = jnp.min(a, axis=0, keepdims=True)
    ahi = jnp.max(a, axis=0, keepdims=True)
    blo = jnp.min(b, axis=0, keepdims=True)
    bhi = jnp.max(b, axis=0, keepdims=True)
    cur = jnp.concatenate([alo, ahi, blo, bhi], axis=0)

    @pl.when(i == 0)
    def _():
        acc_ref[0:4, :] = cur

    @pl.when(i > 0)
    def _():
        old = acc_ref[0:4, :]
        lo = jnp.minimum(old, cur)
        hi = jnp.maximum(old, cur)
        sel = lax.broadcasted_iota(jnp.int32, (4, 64), 0) % 2
        acc_ref[0:4, :] = jnp.where(sel == 0, lo, hi)

    @pl.when(jnp.logical_and(h == 0, i == 0))
    def _():
        g_ref[...] = jnp.zeros((1, 128), jnp.float32)

    @pl.when(i == GN - 1)
    def _():
        lane = lax.broadcasted_iota(jnp.int32, (1, 128), 1)
        w = w2g_ref[0]
        t = jnp.maximum(w * _leaky(acc_ref[1:2, :] + acc_ref[3:4, :]),
                        w * _leaky(acc_ref[0:1, :] + acc_ref[2:3, :]))
        g_ref[...] = g_ref[...] + jnp.sum(t) * jnp.where(lane == h, 1.0, 0.0)


def _gbound(TS, TN, w2g3):
    return pl.pallas_call(
        _gbound_body,
        grid=(3, GN),
        in_specs=[
            pl.BlockSpec((NB, TSW), lambda h, i: (h * GN + i, 0)),
            pl.BlockSpec((NB, TNW), lambda h, i: (h * GN + i, 0)),
            pl.BlockSpec((1, 1, 64), lambda h, i: (h, 0, 0)),
        ],
        out_specs=pl.BlockSpec((1, 128), lambda h, i: (0, 0)),
        out_shape=jax.ShapeDtypeStruct((1, 128), jnp.float32),
        scratch_shapes=[pltpu.VMEM((8, 64), jnp.float32)],
    )(TS, TN, w2g3)


def _make_edge_body(hoff):
    def _edge_body(ts_hbm, tn_hbm, sidx_hbm, nidx_hbm, w2g_hbm, g_hbm,
                   out_hbm, idx_sa, idx_ga, idx_na, idx_sb, idx_gb, idx_nb,
                   rows_sa, rows_na, rows_sb, rows_nb, stage,
                   wv, gv, zbuf, acc, sem_a, sem_b):
        c = lax.axis_index("c")
        s = lax.axis_index("s")
        wid = s * 2 + c
        lane0 = lax.iota(jnp.int32, 16) == 0

        def _zrow(r, _):
            for k in range(PACC // 16):
                zbuf[r, pl.ds(k * 16, 16)] = jnp.zeros((16,), jnp.float32)
            return 0

        lax.fori_loop(0, 40, _zrow, 0)
        nchunks = N // 40
        for j in range(16):
            chunk = s + 16 * j

            @pl.when(chunk < nchunks)
            def _():
                pltpu.sync_copy(zbuf, acc.at[pl.ds(chunk * 40, 40)])

        pltpu.sync_copy(w2g_hbm, wv)
        pltpu.sync_copy(g_hbm, gv)
        gsh = gv[...][0]
        wk = [wv[pl.ds(k * 16, 16)] for k in range(4)]
        plsc.subcore_barrier()

        def _load_idx(bi, idx_s, idx_g, idx_n):
            base = wid * EPW + bi * EB
            pltpu.sync_copy(sidx_hbm.at[pl.ds(base, EB)], idx_s)
            pltpu.sync_copy(nidx_hbm.at[pl.ds(base, EB)], idx_n)
            for k in range(EB // 16):
                idx_g[pl.ds(k * 16, 16)] = idx_s[pl.ds(k * 16, 16)] + hoff
                idx_n[pl.ds(k * 16, 16)] = idx_n[pl.ds(k * 16, 16)] + hoff

        def _start(idx_g, idx_n, rs, rn, sem):
            pltpu.async_copy(ts_hbm.at[idx_g], rs, sem)
            pltpu.async_copy(tn_hbm.at[idx_n], rn, sem)

        def _wait(idx_g, idx_n, rs, rn, sem):
            pltpu.make_async_copy(ts_hbm.at[idx_g], rs, sem).wait()
            pltpu.make_async_copy(tn_hbm.at[idx_n], rn, sem).wait()

        def _compute(rows_s, rows_n, idx_s):
            @plsc.parallel_loop(0, EB, step=1, unroll=4)
            def _edge(e):
                lwv = rows_n[e, pl.ds(TSW, 16)]
                pacc = jnp.zeros((16,), jnp.float32)
                for k in range(4):
                    off = k * 16
                    x = rows_s[e, pl.ds(off, 16)] + rows_n[e, pl.ds(off, 16)]
                    hg = jnp.where(x >= 0, x, 0.01 * x)
                    pacc = pacc + hg * wk[k]
                targ = jnp.sum(pacc) + lwv[0] - gsh
                tv = jnp.exp(jnp.full((16,), targ, jnp.float32))
                for k in range(4):
                    off = 64 + k * 16
                    x = rows_s[e, pl.ds(off, 16)] + rows_n[e, pl.ds(off, 16)]
                    hm = jnp.where(x >= 0, x, 0.01 * x)
                    stage[e, pl.ds(k * 16, 16)] = tv * hm
                stage[e, pl.ds(64, 16)] = jnp.where(lane0, tv, 0.0)

            pltpu.sync_copy(stage, acc.at[idx_s], add=True)

        _load_idx(0, idx_sa, idx_ga, idx_na)
        _start(idx_ga, idx_na, rows_sa, rows_na, sem_a)

        def _pair(i, _):
            _load_idx(2 * i + 1, idx_sb, idx_gb, idx_nb)
            _start(idx_gb, idx_nb, rows_sb, rows_nb, sem_b)
            _wait(idx_ga, idx_na, rows_sa, rows_na, sem_a)
            _compute(rows_sa, rows_na, idx_sa)
            _load_idx(2 * i + 2, idx_sa, idx_ga, idx_na)
            _start(idx_ga, idx_na, rows_sa, rows_na, sem_a)
            _wait(idx_gb, idx_nb, rows_sb, rows_nb, sem_b)
            _compute(rows_sb, rows_nb, idx_sb)
            return 0

        lax.fori_loop(0, (NBATCH - 1) // 2, _pair, 0)
        _wait(idx_ga, idx_na, rows_sa, rows_na, sem_a)
        _compute(rows_sa, rows_na, idx_sa)
        plsc.subcore_barrier()

        for j in range(16):
            chunk = s + 16 * j

            @pl.when(chunk < nchunks)
            def _():
                pltpu.sync_copy(acc.at[pl.ds(chunk * 40, 40)], zbuf)
                pltpu.sync_copy(zbuf, out_hbm.at[c, pl.ds(chunk * 40, 40)])

    return _edge_body


def _edge_pass(TS, TN, sidx, nidx, w2g64, g16, hoff):
    mesh = plsc.VectorSubcoreMesh(core_axis_name="c", subcore_axis_name="s")
    fn = functools.partial(
        pl.kernel,
        mesh=mesh,
        compiler_params=pltpu.CompilerParams(
            use_tc_tiling_on_sc=False, needs_layout_passes=False),
        out_type=jax.ShapeDtypeStruct((2, N, PACC), jnp.float32),
        scratch_types=[
            pltpu.VMEM((EB,), jnp.int32),
            pltpu.VMEM((EB,), jnp.int32),
            pltpu.VMEM((EB,), jnp.int32),
            pltpu.VMEM((EB,), jnp.int32),
            pltpu.VMEM((EB,), jnp.int32),
            pltpu.VMEM((EB,), jnp.int32),
            pltpu.VMEM((EB, TSW), jnp.float32),
            pltpu.VMEM((EB, TNW), jnp.float32),
            pltpu.VMEM((EB, TSW), jnp.float32),
            pltpu.VMEM((EB, TNW), jnp.float32),
            pltpu.VMEM((EB, PACC), jnp.float32),
            pltpu.VMEM((F,), jnp.float32),
            pltpu.VMEM((16,), jnp.float32),
            pltpu.VMEM((40, PACC), jnp.float32),
            pltpu.VMEM_SHARED((N, PACC), jnp.float32),
            pltpu.SemaphoreType.DMA,
            pltpu.SemaphoreType.DMA,
        ],
    )(_make_edge_body(hoff))
    return fn(TS, TN, sidx, nidx, w2g64, g16)


def _combine_body(p0_ref, p1_ref, p2_ref, fea_ref, w2bd_ref, b2_ref,
                  out_ref):
    ps = [p0_ref[0] + p0_ref[1], p1_ref[0] + p1_ref[1], p2_ref[0] + p2_ref[1]]
    numer = jnp.concatenate([p[:, :F] for p in ps], axis=1)
    msgs = jnp.dot(numer, w2bd_ref[...], preferred_element_type=jnp.float32)
    tot = jnp.zeros((NB, F), jnp.float32)
    for h in range(3):
        denom = ps[h][:, F:F + 1]
        o = (msgs[:, h * F:(h + 1) * F]
             + denom * b2_ref[0:1, h * F:(h + 1) * F])
        tot = tot + o / (denom + 1e-30)
    out_ref[...] = fea_ref[...] + tot * (1.0 / 3.0)


def _combine(P0, P1, P2, fea, W2bd, b2cat):
    pspec = pl.BlockSpec((2, NB, PACC), lambda i: (0, i, 0))
    return pl.pallas_call(
        _combine_body,
        grid=(GN,),
        in_specs=[
            pspec, pspec, pspec,
            pl.BlockSpec((NB, F), lambda i: (i, 0)),
            pl.BlockSpec((192, 192), lambda i: (0, 0)),
            pl.BlockSpec((1, 192), lambda i: (0, 0)),
        ],
        out_specs=pl.BlockSpec((NB, F), lambda i: (i, 0)),
        out_shape=jax.ShapeDtypeStruct((N, F), jnp.float32),
    )(P0, P1, P2, fea, W2bd, b2cat)


def _crynode_body(fea_ref, logw_ref, w1g_ref, b1g_ref, w2gp_ref, b2gp_ref,
                  powp_ref, w1m_ref, b1m_ref, x_ref):
    fea = fea_ref[...]
    hg = _leaky(jnp.dot(fea, w1g_ref[...], preferred_element_type=jnp.float32)
                + b1g_ref[...])
    logit = (jnp.dot(hg, w2gp_ref[...], preferred_element_type=jnp.float32)
             + b2gp_ref[...] + logw_ref[...] * powp_ref[...])
    hm = _leaky(jnp.dot(fea, w1m_ref[...], preferred_element_type=jnp.float32)
                + b1m_ref[...])
    x_ref[:, :192] = hm
    x_ref[:, 192:ACCW] = logit[:, 0:16]


def _crynode(fea, logw, W1g_cat, b1g_cat, W2g_pad, b2g_pad, pow_pad,
             W1m_cat, b1m_cat):
    return pl.pallas_call(
        _crynode_body,
        grid=(GN,),
        in_specs=[
            pl.BlockSpec((NB, F), lambda i: (i, 0)),
            pl.BlockSpec((NB, 1), lambda i: (i, 0)),
            pl.BlockSpec((F, 192), lambda i: (0, 0)),
            pl.BlockSpec((1, 192), lambda i: (0, 0)),
            pl.BlockSpec((192, 128), lambda i: (0, 0)),
            pl.BlockSpec((1, 128), lambda i: (0, 0)),
            pl.BlockSpec((1, 128), lambda i: (0, 0)),
            pl.BlockSpec((F, 192), lambda i: (0, 0)),
            pl.BlockSpec((1, 192), lambda i: (0, 0)),
        ],
        out_specs=pl.BlockSpec((NB, ACCW), lambda i: (i, 0)),
        out_shape=jax.ShapeDtypeStruct((N, ACCW), jnp.float32),
    )(fea, logw, W1g_cat, b1g_cat, W2g_pad, b2g_pad, pow_pad,
      W1m_cat, b1m_cat)


def _crymax_body(x_ref, cry_ref, gm_ref):
    ci = pl.program_id(0)
    ni = pl.program_id(1)
    cids = ci * CB + lax.broadcasted_iota(jnp.int32, (NB, CB), 1)
    mask = cry_ref[...] == cids
    cur = jnp.full((8, CB), -1e30, jnp.float32)
    row = lax.broadcasted_iota(jnp.int32, (8, CB), 0)
    for h in range(3):
        lh = x_ref[:, 192 + h:193 + h]
        mh = jnp.max(jnp.where(mask, lh, -1e30), axis=0, keepdims=True)
        cur = jnp.where(row == h, mh, cur)

    @pl.when(ni == 0)
    def _():
        gm_ref[...] = cur

    @pl.when(ni > 0)
    def _():
        gm_ref[...] = jnp.maximum(gm_ref[...], cur)


def _crymax(X, cry2d):
    return pl.pallas_call(
        _crymax_body,
        grid=(GC, GN),
        in_specs=[
            pl.BlockSpec((NB, ACCW), lambda ci, ni: (ni, 0)),
            pl.BlockSpec((NB, 1), lambda ci, ni: (ni, 0)),
        ],
        out_specs=pl.BlockSpec((8, CB), lambda ci, ni: (0, ci)),
        out_shape=jax.ShapeDtypeStruct((8, GC * CB), jnp.float32),
    )(X, cry2d)


def _crysum_body(x_ref, cry_ref, gm_ref, num_ref, d_ref):
    ci = pl.program_id(0)
    ni = pl.program_id(1)
    cids = ci * CB + lax.broadcasted_iota(jnp.int32, (NB, CB), 1)
    mask = cry_ref[...] == cids
    nums = []
    d128 = jnp.zeros((CB, CB), jnp.float32)
    lane = lax.broadcasted_iota(jnp.int32, (CB, CB), 1)
    for h in range(3):
        lh = x_ref[:, 192 + h:193 + h]
        gmh = gm_ref[h:h + 1, :]
        t = jnp.where(mask, jnp.exp(lh - gmh), 0.0)
        hm = x_ref[:, h * 64:(h + 1) * 64]
        nums.append(lax.dot_general(t, hm, (((0,), (0,)), ((), ())),
                                    preferred_element_type=jnp.float32))
        sh = jnp.sum(t, axis=0)
        d128 = d128 + sh[:, None] * jnp.where(lane == h, 1.0, 0.0)
    cur = jnp.concatenate(nums, axis=1)

    @pl.when(ni == 0)
    def _():
        num_ref[...] = cur
        d_ref[...] = d128

    @pl.when(ni > 0)
    def _():
        num_ref[...] = num_ref[...] + cur
        d_ref[...] = d_ref[...] + d128


def _crysum(X, cry2d, GM):
    return pl.pallas_call(
        _crysum_body,
        grid=(GC, GN),
        in_specs=[
            pl.BlockSpec((NB, ACCW), lambda ci, ni: (ni, 0)),
            pl.BlockSpec((NB, 1), lambda ci, ni: (ni, 0)),
            pl.BlockSpec((8, CB), lambda ci, ni: (0, ci)),
        ],
        out_specs=[
            pl.BlockSpec((CB, 192), lambda ci, ni: (ci, 0)),
            pl.BlockSpec((CB, CB), lambda ci, ni: (ci, 0)),
        ],
        out_shape=[
            jax.ShapeDtypeStruct((GC * CB, 192), jnp.float32),
            jax.ShapeDtypeStruct((GC * CB, CB), jnp.float32),
        ],
    )(X, cry2d, GM)


def _cryfin_body(num_ref, d_ref, w2bd_ref, b2_ref, out_ref):
    msgs = jnp.dot(num_ref[...], w2bd_ref[...],
                   preferred_element_type=jnp.float32)
    tot = jnp.zeros((C, F), jnp.float32)
    for h in range(3):
        denom = d_ref[:, h:h + 1]
        o = (msgs[:, h * 64:(h + 1) * 64]
             + denom * b2_ref[0:1, h * 64:(h + 1) * 64])
        tot = tot + o / (denom + 1e-30)
    out_ref[...] = tot * (1.0 / 3.0)


def _cryfin(NUM, D, W2bd, b2cat):
    return pl.pallas_call(
        _cryfin_body,
        grid=(1,),
        in_specs=[
            pl.BlockSpec((C, 192), lambda i: (0, 0)),
            pl.BlockSpec((C, CB), lambda i: (0, 0)),
            pl.BlockSpec((192, 192), lambda i: (0, 0)),
            pl.BlockSpec((1, 192), lambda i: (0, 0)),
        ],
        out_specs=pl.BlockSpec((C, F), lambda i: (0, 0)),
        out_shape=jax.ShapeDtypeStruct((C, F), jnp.float32),
    )(NUM[:C], D[:C], W2bd, b2cat)


def kernel(elem_weights, elem_fea, params, self_fea_idx, nbr_fea_idx,
           cry_elem_idx):
    Wemb, bemb = params["emb"]
    W64 = jnp.pad(Wemb, ((0, 0), (0, 1)))
    b64 = jnp.pad(bemb, (0, 1)).reshape(1, F)
    fea, logw = _embed(elem_fea, elem_weights, W64, b64)

    sidx = self_fea_idx.astype(jnp.int32)
    nidx = nbr_fea_idx.astype(jnp.int32)

    for heads in params["graphs"]:
        Ws_l, bs_l, Wn_l, pow_l, w2g_l, W2m, b2m = [], [], [], [], [], [], []
        for p in heads:
            (W1g, b1g), (w2g_, _) = p["gate"]
            (W1m, b1m), (W2m_, b2m_) = p["msg"]
            Ws_l.append(jnp.concatenate([W1g[:F], W1m[:F]], axis=1))
            bs_l.append(jnp.concatenate([b1g, b1m]).reshape(1, TSW))
            Wn_l.append(jnp.concatenate([W1g[F:], W1m[F:]], axis=1))
            pw = jnp.zeros((1, 128), jnp.float32).at[0, 0].set(p["pow"][0])
            pow_l.append(pw)
            w2g_l.append(w2g_[:, 0].reshape(1, F))
            W2m.append(W2m_)
            b2m.append(b2m_)
        Ws3 = jnp.stack(Ws_l)
        bs3 = jnp.stack(bs_l)
        Wn3 = jnp.stack(Wn_l)
        pow3 = jnp.stack(pow_l)
        w2g3 = jnp.stack(w2g_l)
        TS, TN = _tables(fea, logw, Ws3, bs3, Wn3, pow3)
        G = _gbound(TS, TN, w2g3)

        Ps = []
        for h in range(3):
            g16 = jnp.broadcast_to(G[0, h], (16,))
            Ps.append(_edge_pass(TS, TN, sidx, nidx, w2g3[h, 0], g16,
                                 h * N))
        W2bd = jax.scipy.linalg.block_diag(*W2m)
        b2cat = jnp.concatenate(b2m).reshape(1, 192)
        fea = _combine(Ps[0], Ps[1], Ps[2], fea, W2bd, b2cat)

    W1g_cat = jnp.concatenate([p["gate"][0][0] for p in params["cry"]],
                              axis=1)
    b1g_cat = jnp.concatenate([p["gate"][0][1] for p in params["cry"]]
                              ).reshape(1, 192)
    W2g_bd = jax.scipy.linalg.block_diag(
        *[p["gate"][1][0] for p in params["cry"]])
    W2g_pad = jnp.pad(W2g_bd, ((0, 0), (0, 125)))
    b2g_pad = jnp.zeros((1, 128), jnp.float32).at[0, :3].set(
        jnp.stack([p["gate"][1][1][0] for p in params["cry"]]))
    pow_pad = jnp.zeros((1, 128), jnp.float32).at[0, :3].set(
        jnp.stack([p["pow"][0] for p in params["cry"]]))
    W1m_cat = jnp.concatenate([p["msg"][0][0] for p in params["cry"]],
                              axis=1)
    b1m_cat = jnp.concatenate([p["msg"][0][1] for p in params["cry"]]
                              ).reshape(1, 192)
    X = _crynode(fea, logw, W1g_cat, b1g_cat, W2g_pad, b2g_pad, pow_pad,
                 W1m_cat, b1m_cat)
    cry2d = cry_elem_idx.astype(jnp.int32).reshape(N, 1)
    GM = _crymax(X, cry2d)
    NUM, D = _crysum(X, cry2d, GM)
    W2bd_c = jax.scipy.linalg.block_diag(
        *[p["msg"][1][0] for p in params["cry"]])
    b2cat_c = jnp.concatenate([p["msg"][1][1] for p in params["cry"]]
                              ).reshape(1, 192)
    return _cryfin(NUM, D, W2bd_c, b2cat_c)


# speedup vs baseline: 19.9412x; 1.2784x over previous
import functools

import jax
import jax.numpy as jnp
from jax import lax
from jax.experimental import pallas as pl
from jax.experimental.pallas import tpu as pltpu
from jax.experimental.pallas import tpu_sc as plsc

N = 10000
M = 320000
C = 1000
F = 64
NB = 1000
GN = N // NB
CB = 128
GC = (C + CB - 1) // CB
ACCW = 208
TSW = 128
TNW = 144
PACC = 80
NW = 32
EPW = M // NW
EB = 80
NBATCH = EPW // EB


def _leaky(x):
    return jnp.where(x >= 0, x, 0.01 * x)


def _emb_body(ef_ref, w_ref, b_ref, ew_ref, fea_ref, logw_ref):
    y = jnp.dot(ef_ref[...], w_ref[...], preferred_element_type=jnp.float32)
    y = y + b_ref[...]
    ew = ew_ref[...]
    mask = lax.broadcasted_iota(jnp.int32, (NB, F), 1) == (F - 1)
    fea_ref[...] = y + jnp.where(mask, ew, 0.0)
    logw_ref[...] = jnp.log(ew)


def _embed(elem_fea, elem_weights, W64, b64):
    return pl.pallas_call(
        _emb_body,
        grid=(GN,),
        in_specs=[
            pl.BlockSpec((NB, 200), lambda i: (i, 0)),
            pl.BlockSpec((200, F), lambda i: (0, 0)),
            pl.BlockSpec((1, F), lambda i: (0, 0)),
            pl.BlockSpec((NB, 1), lambda i: (i, 0)),
        ],
        out_specs=[
            pl.BlockSpec((NB, F), lambda i: (i, 0)),
            pl.BlockSpec((NB, 1), lambda i: (i, 0)),
        ],
        out_shape=[
            jax.ShapeDtypeStruct((N, F), jnp.float32),
            jax.ShapeDtypeStruct((N, 1), jnp.float32),
        ],
    )(elem_fea, W64, b64, elem_weights)


def _tables_body(fea_ref, logw_ref, ws_ref, bs_ref, wn_ref, pow_ref,
                 ts_ref, tn_ref):
    fea = fea_ref[...]
    ts_ref[...] = (
        jnp.dot(fea, ws_ref[0], preferred_element_type=jnp.float32)
        + bs_ref[0])
    tn_ref[:, :TSW] = jnp.dot(fea, wn_ref[0],
                              preferred_element_type=jnp.float32)
    tn_ref[:, TSW:TNW] = logw_ref[...] * pow_ref[0, 0:1, 0:16]


def _tables(fea, logw, Ws3, bs3, Wn3, pow3):
    return pl.pallas_call(
        _tables_body,
        grid=(3, GN),
        in_specs=[
            pl.BlockSpec((NB, F), lambda h, i: (i, 0)),
            pl.BlockSpec((NB, 1), lambda h, i: (i, 0)),
            pl.BlockSpec((1, F, TSW), lambda h, i: (h, 0, 0)),
            pl.BlockSpec((1, 1, TSW), lambda h, i: (h, 0, 0)),
            pl.BlockSpec((1, F, TSW), lambda h, i: (h, 0, 0)),
            pl.BlockSpec((1, 1, 128), lambda h, i: (h, 0, 0)),
        ],
        out_specs=[
            pl.BlockSpec((NB, TSW), lambda h, i: (h * GN + i, 0)),
            pl.BlockSpec((NB, TNW), lambda h, i: (h * GN + i, 0)),
        ],
        out_shape=[
            jax.ShapeDtypeStruct((3 * N, TSW), jnp.float32),
            jax.ShapeDtypeStruct((3 * N, TNW), jnp.float32),
        ],
    )(fea, logw, Ws3, bs3, Wn3, pow3)


def _gbound_body(ts_ref, tn_ref, w2g_ref, g_ref, acc_ref):
    h = pl.program_id(0)
    i = pl.program_id(1)
    a = ts_ref[:, :64]
    b = tn_ref[:, :64]
    alo = jnp.min(a, axis=0, keepdims=True)
    ahi = jnp.max(a, axis=0, keepdims=True)
    blo = jnp.min(b, axis=0, keepdims=True)
    bhi = jnp.max(b, axis=0, keepdims=True)
    cur = jnp.concatenate([alo, ahi, blo, bhi], axis=0)

    @pl.when(i == 0)
    def _():
        acc_ref[0:4, :] = cur

    @pl.when(i > 0)
    def _():
        old = acc_ref[0:4, :]
        lo = jnp.minimum(old, cur)
        hi = jnp.maximum(old, cur)
        sel = lax.broadcasted_iota(jnp.int32, (4, 64), 0) % 2
        acc_ref[0:4, :] = jnp.where(sel == 0, lo, hi)

    @pl.when(jnp.logical_and(h == 0, i == 0))
    def _():
        g_ref[...] = jnp.zeros((1, 128), jnp.float32)

    @pl.when(i == GN - 1)
    def _():
        lane = lax.broadcasted_iota(jnp.int32, (1, 128), 1)
        w = w2g_ref[0]
        t = jnp.maximum(w * _leaky(acc_ref[1:2, :] + acc_ref[3:4, :]),
                        w * _leaky(acc_ref[0:1, :] + acc_ref[2:3, :]))
        g_ref[...] = g_ref[...] + jnp.sum(t) * jnp.where(lane == h, 1.0, 0.0)


def _gbound(TS, TN, w2g3):
    return pl.pallas_call(
        _gbound_body,
        grid=(3, GN),
        in_specs=[
            pl.BlockSpec((NB, TSW), lambda h, i: (h * GN + i, 0)),
            pl.BlockSpec((NB, TNW), lambda h, i: (h * GN + i, 0)),
            pl.BlockSpec((1, 1, 64), lambda h, i: (h, 0, 0)),
        ],
        out_specs=pl.BlockSpec((1, 128), lambda h, i: (0, 0)),
        out_shape=jax.ShapeDtypeStruct((1, 128), jnp.float32),
        scratch_shapes=[pltpu.VMEM((8, 64), jnp.float32)],
    )(TS, TN, w2g3)


def _make_edge_body(hoff):
    def _edge_body(ts_hbm, tn_hbm, sidx_hbm, nidx_hbm, w2g_hbm, g_hbm,
                   out_hbm, idx_s_all, idx_n_all,
                   idx_sa, idx_ga, idx_na, idx_sb, idx_gb, idx_nb,
                   rows_sa, rows_na, rows_sb, rows_nb, stage,
                   wv, gv, zbuf, acc, sem_a, sem_b):
        c = lax.axis_index("c")
        s = lax.axis_index("s")
        wid = s * 2 + c
        lane0 = lax.iota(jnp.int32, 16) == 0

        def _zrow(r, _):
            for k in range(PACC // 16):
                zbuf[r, pl.ds(k * 16, 16)] = jnp.zeros((16,), jnp.float32)
            return 0

        lax.fori_loop(0, 40, _zrow, 0)
        nchunks = N // 40
        for j in range(16):
            chunk = s + 16 * j

            @pl.when(chunk < nchunks)
            def _():
                pltpu.sync_copy(zbuf, acc.at[pl.ds(chunk * 40, 40)])

        pltpu.sync_copy(w2g_hbm, wv)
        pltpu.sync_copy(g_hbm, gv)
        gsh = gv[...][0]
        wk = [wv[pl.ds(k * 16, 16)] for k in range(4)]
        plsc.subcore_barrier()

        pltpu.sync_copy(sidx_hbm.at[pl.ds(wid * EPW, EPW)], idx_s_all)
        pltpu.sync_copy(nidx_hbm.at[pl.ds(wid * EPW, EPW)], idx_n_all)

        def _load_idx(bi, idx_s, idx_g, idx_n):
            for k in range(EB // 16):
                sl = pl.ds(bi * EB + k * 16, 16)
                v = idx_s_all[sl]
                idx_s[pl.ds(k * 16, 16)] = v
                idx_g[pl.ds(k * 16, 16)] = v + hoff
                idx_n[pl.ds(k * 16, 16)] = idx_n_all[sl] + hoff

        def _start(idx_g, idx_n, rs, rn, sem):
            pltpu.async_copy(ts_hbm.at[idx_g], rs, sem)
            pltpu.async_copy(tn_hbm.at[idx_n], rn, sem)

        def _wait(idx_g, idx_n, rs, rn, sem):
            pltpu.make_async_copy(ts_hbm.at[idx_g], rs, sem).wait()
            pltpu.make_async_copy(tn_hbm.at[idx_n], rn, sem).wait()

        def _compute(rows_s, rows_n, idx_s):
            @plsc.parallel_loop(0, EB, step=1, unroll=8)
            def _edge(e):
                lwv = rows_n[e, pl.ds(TSW, 16)]
                pacc = jnp.zeros((16,), jnp.float32)
                for k in range(4):
                    off = k * 16
                    x = rows_s[e, pl.ds(off, 16)] + rows_n[e, pl.ds(off, 16)]
                    pacc = pacc + jnp.maximum(x, 0.01 * x) * wk[k]
                targ = jnp.sum(pacc) + lwv[0] - gsh
                tv = jnp.exp(jnp.full((16,), targ, jnp.float32))
                for k in range(4):
                    off = 64 + k * 16
                    x = rows_s[e, pl.ds(off, 16)] + rows_n[e, pl.ds(off, 16)]
                    stage[e, pl.ds(k * 16, 16)] = tv * jnp.maximum(x, 0.01 * x)
                stage[e, pl.ds(64, 16)] = jnp.where(lane0, tv, 0.0)

            pltpu.sync_copy(stage, acc.at[idx_s], add=True)

        _load_idx(0, idx_sa, idx_ga, idx_na)
        _start(idx_ga, idx_na, rows_sa, rows_na, sem_a)

        def _pair(i, _):
            _load_idx(2 * i + 1, idx_sb, idx_gb, idx_nb)
            _start(idx_gb, idx_nb, rows_sb, rows_nb, sem_b)
            _wait(idx_ga, idx_na, rows_sa, rows_na, sem_a)
            _compute(rows_sa, rows_na, idx_sa)
            _load_idx(2 * i + 2, idx_sa, idx_ga, idx_na)
            _start(idx_ga, idx_na, rows_sa, rows_na, sem_a)
            _wait(idx_gb, idx_nb, rows_sb, rows_nb, sem_b)
            _compute(rows_sb, rows_nb, idx_sb)
            return 0

        lax.fori_loop(0, (NBATCH - 1) // 2, _pair, 0)
        _wait(idx_ga, idx_na, rows_sa, rows_na, sem_a)
        _compute(rows_sa, rows_na, idx_sa)
        plsc.subcore_barrier()

        for j in range(16):
            chunk = s + 16 * j

            @pl.when(chunk < nchunks)
            def _():
                pltpu.sync_copy(acc.at[pl.ds(chunk * 40, 40)], zbuf)
                pltpu.sync_copy(zbuf, out_hbm.at[c, pl.ds(chunk * 40, 40)])

    return _edge_body


def _edge_pass(TS, TN, sidx, nidx, w2g64, g16, hoff):
    mesh = plsc.VectorSubcoreMesh(core_axis_name="c", subcore_axis_name="s")
    fn = functools.partial(
        pl.kernel,
        mesh=mesh,
        compiler_params=pltpu.CompilerParams(
            use_tc_tiling_on_sc=False, needs_layout_passes=False),
        out_type=jax.ShapeDtypeStruct((2, N, PACC), jnp.float32),
        scratch_types=[
            pltpu.VMEM((EPW,), jnp.int32),
            pltpu.VMEM((EPW,), jnp.int32),
            pltpu.VMEM((EB,), jnp.int32),
            pltpu.VMEM((EB,), jnp.int32),
            pltpu.VMEM((EB,), jnp.int32),
            pltpu.VMEM((EB,), jnp.int32),
            pltpu.VMEM((EB,), jnp.int32),
            pltpu.VMEM((EB,), jnp.int32),
            pltpu.VMEM((EB, TSW), jnp.float32),
            pltpu.VMEM((EB, TNW), jnp.float32),
            pltpu.VMEM((EB, TSW), jnp.float32),
            pltpu.VMEM((EB, TNW), jnp.float32),
            pltpu.VMEM((EB, PACC), jnp.float32),
            pltpu.VMEM((F,), jnp.float32),
            pltpu.VMEM((16,), jnp.float32),
            pltpu.VMEM((40, PACC), jnp.float32),
            pltpu.VMEM_SHARED((N, PACC), jnp.float32),
            pltpu.SemaphoreType.DMA,
            pltpu.SemaphoreType.DMA,
        ],
    )(_make_edge_body(hoff))
    return fn(TS, TN, sidx, nidx, w2g64, g16)


def _combine_body(p0_ref, p1_ref, p2_ref, fea_ref, w2bd_ref, b2_ref,
                  out_ref):
    ps = [p0_ref[0] + p0_ref[1], p1_ref[0] + p1_ref[1], p2_ref[0] + p2_ref[1]]
    numer = jnp.concatenate([p[:, :F] for p in ps], axis=1)
    msgs = jnp.dot(numer, w2bd_ref[...], preferred_element_type=jnp.float32)
    tot = jnp.zeros((NB, F), jnp.float32)
    for h in range(3):
        denom = ps[h][:, F:F + 1]
        o = (msgs[:, h * F:(h + 1) * F]
             + denom * b2_ref[0:1, h * F:(h + 1) * F])
        tot = tot + o / (denom + 1e-30)
    out_ref[...] = fea_ref[...] + tot * (1.0 / 3.0)


def _combine(P0, P1, P2, fea, W2bd, b2cat):
    pspec = pl.BlockSpec((2, NB, PACC), lambda i: (0, i, 0))
    return pl.pallas_call(
        _combine_body,
        grid=(GN,),
        in_specs=[
            pspec, pspec, pspec,
            pl.BlockSpec((NB, F), lambda i: (i, 0)),
            pl.BlockSpec((192, 192), lambda i: (0, 0)),
            pl.BlockSpec((1, 192), lambda i: (0, 0)),
        ],
        out_specs=pl.BlockSpec((NB, F), lambda i: (i, 0)),
        out_shape=jax.ShapeDtypeStruct((N, F), jnp.float32),
    )(P0, P1, P2, fea, W2bd, b2cat)


def _crynode_body(fea_ref, logw_ref, w1g_ref, b1g_ref, w2gp_ref, b2gp_ref,
                  powp_ref, w1m_ref, b1m_ref, x_ref):
    fea = fea_ref[...]
    hg = _leaky(jnp.dot(fea, w1g_ref[...], preferred_element_type=jnp.float32)
                + b1g_ref[...])
    logit = (jnp.dot(hg, w2gp_ref[...], preferred_element_type=jnp.float32)
             + b2gp_ref[...] + logw_ref[...] * powp_ref[...])
    hm = _leaky(jnp.dot(fea, w1m_ref[...], preferred_element_type=jnp.float32)
                + b1m_ref[...])
    x_ref[:, :192] = hm
    x_ref[:, 192:ACCW] = logit[:, 0:16]


def _crynode(fea, logw, W1g_cat, b1g_cat, W2g_pad, b2g_pad, pow_pad,
             W1m_cat, b1m_cat):
    return pl.pallas_call(
        _crynode_body,
        grid=(GN,),
        in_specs=[
            pl.BlockSpec((NB, F), lambda i: (i, 0)),
            pl.BlockSpec((NB, 1), lambda i: (i, 0)),
            pl.BlockSpec((F, 192), lambda i: (0, 0)),
            pl.BlockSpec((1, 192), lambda i: (0, 0)),
            pl.BlockSpec((192, 128), lambda i: (0, 0)),
            pl.BlockSpec((1, 128), lambda i: (0, 0)),
            pl.BlockSpec((1, 128), lambda i: (0, 0)),
            pl.BlockSpec((F, 192), lambda i: (0, 0)),
            pl.BlockSpec((1, 192), lambda i: (0, 0)),
        ],
        out_specs=pl.BlockSpec((NB, ACCW), lambda i: (i, 0)),
        out_shape=jax.ShapeDtypeStruct((N, ACCW), jnp.float32),
    )(fea, logw, W1g_cat, b1g_cat, W2g_pad, b2g_pad, pow_pad,
      W1m_cat, b1m_cat)


def _crymax_body(x_ref, cry_ref, gm_ref):
    ci = pl.program_id(0)
    ni = pl.program_id(1)
    cids = ci * CB + lax.broadcasted_iota(jnp.int32, (NB, CB), 1)
    mask = cry_ref[...] == cids
    cur = jnp.full((8, CB), -1e30, jnp.float32)
    row = lax.broadcasted_iota(jnp.int32, (8, CB), 0)
    for h in range(3):
        lh = x_ref[:, 192 + h:193 + h]
        mh = jnp.max(jnp.where(mask, lh, -1e30), axis=0, keepdims=True)
        cur = jnp.where(row == h, mh, cur)

    @pl.when(ni == 0)
    def _():
        gm_ref[...] = cur

    @pl.when(ni > 0)
    def _():
        gm_ref[...] = jnp.maximum(gm_ref[...], cur)


def _crymax(X, cry2d):
    return pl.pallas_call(
        _crymax_body,
        grid=(GC, GN),
        in_specs=[
            pl.BlockSpec((NB, ACCW), lambda ci, ni: (ni, 0)),
            pl.BlockSpec((NB, 1), lambda ci, ni: (ni, 0)),
        ],
        out_specs=pl.BlockSpec((8, CB), lambda ci, ni: (0, ci)),
        out_shape=jax.ShapeDtypeStruct((8, GC * CB), jnp.float32),
    )(X, cry2d)


def _crysum_body(x_ref, cry_ref, gm_ref, num_ref, d_ref):
    ci = pl.program_id(0)
    ni = pl.program_id(1)
    cids = ci * CB + lax.broadcasted_iota(jnp.int32, (NB, CB), 1)
    mask = cry_ref[...] == cids
    nums = []
    d128 = jnp.zeros((CB, CB), jnp.float32)
    lane = lax.broadcasted_iota(jnp.int32, (CB, CB), 1)
    for h in range(3):
        lh = x_ref[:, 192 + h:193 + h]
        gmh = gm_ref[h:h + 1, :]
        t = jnp.where(mask, jnp.exp(lh - gmh), 0.0)
        hm = x_ref[:, h * 64:(h + 1) * 64]
        nums.append(lax.dot_general(t, hm, (((0,), (0,)), ((), ())),
                                    preferred_element_type=jnp.float32))
        sh = jnp.sum(t, axis=0)
        d128 = d128 + sh[:, None] * jnp.where(lane == h, 1.0, 0.0)
    cur = jnp.concatenate(nums, axis=1)

    @pl.when(ni == 0)
    def _():
        num_ref[...] = cur
        d_ref[...] = d128

    @pl.when(ni > 0)
    def _():
        num_ref[...] = num_ref[...] + cur
        d_ref[...] = d_ref[...] + d128


def _crysum(X, cry2d, GM):
    return pl.pallas_call(
        _crysum_body,
        grid=(GC, GN),
        in_specs=[
            pl.BlockSpec((NB, ACCW), lambda ci, ni: (ni, 0)),
            pl.BlockSpec((NB, 1), lambda ci, ni: (ni, 0)),
            pl.BlockSpec((8, CB), lambda ci, ni: (0, ci)),
        ],
        out_specs=[
            pl.BlockSpec((CB, 192), lambda ci, ni: (ci, 0)),
            pl.BlockSpec((CB, CB), lambda ci, ni: (ci, 0)),
        ],
        out_shape=[
            jax.ShapeDtypeStruct((GC * CB, 192), jnp.float32),
            jax.ShapeDtypeStruct((GC * CB, CB), jnp.float32),
        ],
    )(X, cry2d, GM)


def _cryfin_body(num_ref, d_ref, w2bd_ref, b2_ref, out_ref):
    msgs = jnp.dot(num_ref[...], w2bd_ref[...],
                   preferred_element_type=jnp.float32)
    tot = jnp.zeros((C, F), jnp.float32)
    for h in range(3):
        denom = d_ref[:, h:h + 1]
        o = (msgs[:, h * 64:(h + 1) * 64]
             + denom * b2_ref[0:1, h * 64:(h + 1) * 64])
        tot = tot + o / (denom + 1e-30)
    out_ref[...] = tot * (1.0 / 3.0)


def _cryfin(NUM, D, W2bd, b2cat):
    return pl.pallas_call(
        _cryfin_body,
        grid=(1,),
        in_specs=[
            pl.BlockSpec((C, 192), lambda i: (0, 0)),
            pl.BlockSpec((C, CB), lambda i: (0, 0)),
            pl.BlockSpec((192, 192), lambda i: (0, 0)),
            pl.BlockSpec((1, 192), lambda i: (0, 0)),
        ],
        out_specs=pl.BlockSpec((C, F), lambda i: (0, 0)),
        out_shape=jax.ShapeDtypeStruct((C, F), jnp.float32),
    )(NUM[:C], D[:C], W2bd, b2cat)


def kernel(elem_weights, elem_fea, params, self_fea_idx, nbr_fea_idx,
           cry_elem_idx):
    Wemb, bemb = params["emb"]
    W64 = jnp.pad(Wemb, ((0, 0), (0, 1)))
    b64 = jnp.pad(bemb, (0, 1)).reshape(1, F)
    fea, logw = _embed(elem_fea, elem_weights, W64, b64)

    sidx = self_fea_idx.astype(jnp.int32)
    nidx = nbr_fea_idx.astype(jnp.int32)

    for heads in params["graphs"]:
        Ws_l, bs_l, Wn_l, pow_l, w2g_l, W2m, b2m = [], [], [], [], [], [], []
        for p in heads:
            (W1g, b1g), (w2g_, _) = p["gate"]
            (W1m, b1m), (W2m_, b2m_) = p["msg"]
            Ws_l.append(jnp.concatenate([W1g[:F], W1m[:F]], axis=1))
            bs_l.append(jnp.concatenate([b1g, b1m]).reshape(1, TSW))
            Wn_l.append(jnp.concatenate([W1g[F:], W1m[F:]], axis=1))
            pw = jnp.zeros((1, 128), jnp.float32).at[0, 0].set(p["pow"][0])
            pow_l.append(pw)
            w2g_l.append(w2g_[:, 0].reshape(1, F))
            W2m.append(W2m_)
            b2m.append(b2m_)
        Ws3 = jnp.stack(Ws_l)
        bs3 = jnp.stack(bs_l)
        Wn3 = jnp.stack(Wn_l)
        pow3 = jnp.stack(pow_l)
        w2g3 = jnp.stack(w2g_l)
        TS, TN = _tables(fea, logw, Ws3, bs3, Wn3, pow3)
        G = _gbound(TS, TN, w2g3)

        Ps = []
        for h in range(3):
            g16 = jnp.broadcast_to(G[0, h], (16,))
            Ps.append(_edge_pass(TS, TN, sidx, nidx, w2g3[h, 0], g16,
                                 h * N))
        W2bd = jax.scipy.linalg.block_diag(*W2m)
        b2cat = jnp.concatenate(b2m).reshape(1, 192)
        fea = _combine(Ps[0], Ps[1], Ps[2], fea, W2bd, b2cat)

    W1g_cat = jnp.concatenate([p["gate"][0][0] for p in params["cry"]],
                              axis=1)
    b1g_cat = jnp.concatenate([p["gate"][0][1] for p in params["cry"]]
                              ).reshape(1, 192)
    W2g_bd = jax.scipy.linalg.block_diag(
        *[p["gate"][1][0] for p in params["cry"]])
    W2g_pad = jnp.pad(W2g_bd, ((0, 0), (0, 125)))
    b2g_pad = jnp.zeros((1, 128), jnp.float32).at[0, :3].set(
        jnp.stack([p["gate"][1][1][0] for p in params["cry"]]))
    pow_pad = jnp.zeros((1, 128), jnp.float32).at[0, :3].set(
        jnp.stack([p["pow"][0] for p in params["cry"]]))
    W1m_cat = jnp.concatenate([p["msg"][0][0] for p in params["cry"]],
                              axis=1)
    b1m_cat = jnp.concatenate([p["msg"][0][1] for p in params["cry"]]
                              ).reshape(1, 192)
    X = _crynode(fea, logw, W1g_cat, b1g_cat, W2g_pad, b2g_pad, pow_pad,
                 W1m_cat, b1m_cat)
    cry2d = cry_elem_idx.astype(jnp.int32).reshape(N, 1)
    GM = _crymax(X, cry2d)
    NUM, D = _crysum(X, cry2d, GM)
    W2bd_c = jax.scipy.linalg.block_diag(
        *[p["msg"][1][0] for p in params["cry"]])
    b2cat_c = jnp.concatenate([p["msg"][1][1] for p in params["cry"]]
                              ).reshape(1, 192)
    return _cryfin(NUM, D, W2bd_c, b2cat_c)


# speedup vs baseline: 21.8967x; 1.0981x over previous
import functools

import jax
import jax.numpy as jnp
from jax import lax
from jax.experimental import pallas as pl
from jax.experimental.pallas import tpu as pltpu
from jax.experimental.pallas import tpu_sc as plsc

N = 10000
M = 320000
C = 1000
F = 64
NB = 1000
GN = N // NB
CB = 128
GC = (C + CB - 1) // CB
ACCW = 208
TSW = 128
TNW = 144
PACC = 80
NW = 32
EPW = M // NW
EB = 80
NBATCH = EPW // EB


def _leaky(x):
    return jnp.maximum(x, 0.01 * x)


def _emit_tables(i, fea, logw, ws_ref, bs_ref, wn_ref, pow_ref, w2g_ref,
                 ts_refs, tn_refs, g_ref, mm_ref):
    for h in range(3):
        a = (jnp.dot(fea, ws_ref[h], preferred_element_type=jnp.float32)
             + bs_ref[h])
        b = jnp.dot(fea, wn_ref[h], preferred_element_type=jnp.float32)
        ts_refs[h][...] = a
        tn_refs[h][:, :TSW] = b
        tn_refs[h][:, TSW:TNW] = logw * pow_ref[h, 0:1, 0:16]
        ag, bg = a[:, :64], b[:, :64]
        cur = jnp.concatenate(
            [jnp.min(ag, axis=0, keepdims=True),
             jnp.max(ag, axis=0, keepdims=True),
             jnp.min(bg, axis=0, keepdims=True),
             jnp.max(bg, axis=0, keepdims=True)], axis=0)
        col = pl.ds(h * 64, 64)

        @pl.when(i == 0)
        def _():
            mm_ref[0:4, col] = cur

        @pl.when(i > 0)
        def _():
            old = mm_ref[0:4, col]
            lo = jnp.minimum(old, cur)
            hi = jnp.maximum(old, cur)
            sel = lax.broadcasted_iota(jnp.int32, (4, 64), 0) % 2
            mm_ref[0:4, col] = jnp.where(sel == 0, lo, hi)

    @pl.when(i == GN - 1)
    def _():
        lane = lax.broadcasted_iota(jnp.int32, (1, 128), 1)
        g = jnp.zeros((1, 128), jnp.float32)
        for h in range(3):
            col = pl.ds(h * 64, 64)
            w = w2g_ref[h]
            t = jnp.maximum(
                w * _leaky(mm_ref[1:2, col] + mm_ref[3:4, col]),
                w * _leaky(mm_ref[0:1, col] + mm_ref[2:3, col]))
            g = g + jnp.sum(t) * jnp.where(lane == h, 1.0, 0.0)
        g_ref[...] = g


_TABLE_IN_SPECS = [
    pl.BlockSpec((3, F, TSW), lambda i: (0, 0, 0)),
    pl.BlockSpec((3, 1, TSW), lambda i: (0, 0, 0)),
    pl.BlockSpec((3, F, TSW), lambda i: (0, 0, 0)),
    pl.BlockSpec((3, 1, 128), lambda i: (0, 0, 0)),
    pl.BlockSpec((3, 1, F), lambda i: (0, 0, 0)),
]

_TABLE_OUT_SPECS = (
    [pl.BlockSpec((NB, TSW), lambda i: (i, 0)) for _ in range(3)]
    + [pl.BlockSpec((NB, TNW), lambda i: (i, 0)) for _ in range(3)]
    + [pl.BlockSpec((1, 128), lambda i: (0, 0))]
)

_TABLE_OUT_SHAPES = (
    [jax.ShapeDtypeStruct((N, TSW), jnp.float32) for _ in range(3)]
    + [jax.ShapeDtypeStruct((N, TNW), jnp.float32) for _ in range(3)]
    + [jax.ShapeDtypeStruct((1, 128), jnp.float32)]
)


def _k0_body(ef_ref, w_ref, b_ref, ew_ref, ws_ref, bs_ref, wn_ref, pow_ref,
             w2g_ref, fea_ref, logw_ref, ts0, ts1, ts2, tn0, tn1, tn2,
             g_ref, mm_ref):
    i = pl.program_id(0)
    y = jnp.dot(ef_ref[...], w_ref[...], preferred_element_type=jnp.float32)
    y = y + b_ref[...]
    ew = ew_ref[...]
    mask = lax.broadcasted_iota(jnp.int32, (NB, F), 1) == (F - 1)
    fea = y + jnp.where(mask, ew, 0.0)
    logw = jnp.log(ew)
    fea_ref[...] = fea
    logw_ref[...] = logw
    _emit_tables(i, fea, logw, ws_ref, bs_ref, wn_ref, pow_ref, w2g_ref,
                 [ts0, ts1, ts2], [tn0, tn1, tn2], g_ref, mm_ref)


def _k0(elem_fea, elem_weights, W64, b64, Ws3, bs3, Wn3, pow3, w2g3):
    return pl.pallas_call(
        _k0_body,
        grid=(GN,),
        in_specs=[
            pl.BlockSpec((NB, 200), lambda i: (i, 0)),
            pl.BlockSpec((200, F), lambda i: (0, 0)),
            pl.BlockSpec((1, F), lambda i: (0, 0)),
            pl.BlockSpec((NB, 1), lambda i: (i, 0)),
        ] + _TABLE_IN_SPECS,
        out_specs=[
            pl.BlockSpec((NB, F), lambda i: (i, 0)),
            pl.BlockSpec((NB, 1), lambda i: (i, 0)),
        ] + _TABLE_OUT_SPECS,
        out_shape=[
            jax.ShapeDtypeStruct((N, F), jnp.float32),
            jax.ShapeDtypeStruct((N, 1), jnp.float32),
        ] + _TABLE_OUT_SHAPES,
        scratch_shapes=[pltpu.VMEM((8, 192), jnp.float32)],
    )(elem_fea, W64, b64, elem_weights, Ws3, bs3, Wn3, pow3, w2g3)


def _combine_val(p0, p1, p2, fea, w2bd_ref, b2_ref):
    ps = [p0, p1, p2]
    numer = jnp.concatenate([p[:, :F] for p in ps], axis=1)
    msgs = jnp.dot(numer, w2bd_ref[...], preferred_element_type=jnp.float32)
    tot = jnp.zeros((NB, F), jnp.float32)
    for h in range(3):
        denom = ps[h][:, F:F + 1]
        o = (msgs[:, h * F:(h + 1) * F]
             + denom * b2_ref[0:1, h * F:(h + 1) * F])
        tot = tot + o / (denom + 1e-30)
    return fea + tot * (1.0 / 3.0)


def _kc_body(p0_ref, p1_ref, p2_ref, fea_ref, w2bd_ref, b2_ref, logw_ref,
             ws_ref, bs_ref, wn_ref, pow_ref, w2g_ref,
             feao_ref, ts0, ts1, ts2, tn0, tn1, tn2, g_ref, mm_ref):
    i = pl.program_id(0)
    fea = _combine_val(p0_ref[0] + p0_ref[1], p1_ref[0] + p1_ref[1],
                       p2_ref[0] + p2_ref[1], fea_ref[...], w2bd_ref,
                       b2_ref)
    feao_ref[...] = fea
    _emit_tables(i, fea, logw_ref[...], ws_ref, bs_ref, wn_ref, pow_ref,
                 w2g_ref, [ts0, ts1, ts2], [tn0, tn1, tn2], g_ref, mm_ref)


def _kc(P0, P1, P2, fea, logw, W2bd, b2cat, Ws3, bs3, Wn3, pow3, w2g3):
    pspec = pl.BlockSpec((2, NB, PACC), lambda i: (0, i, 0))
    return pl.pallas_call(
        _kc_body,
        grid=(GN,),
        in_specs=[
            pspec, pspec, pspec,
            pl.BlockSpec((NB, F), lambda i: (i, 0)),
            pl.BlockSpec((192, 192), lambda i: (0, 0)),
            pl.BlockSpec((1, 192), lambda i: (0, 0)),
            pl.BlockSpec((NB, 1), lambda i: (i, 0)),
        ] + _TABLE_IN_SPECS,
        out_specs=[pl.BlockSpec((NB, F), lambda i: (i, 0))]
        + _TABLE_OUT_SPECS,
        out_shape=[jax.ShapeDtypeStruct((N, F), jnp.float32)]
        + _TABLE_OUT_SHAPES,
        scratch_shapes=[pltpu.VMEM((8, 192), jnp.float32)],
    )(P0, P1, P2, fea, W2bd, b2cat, logw, Ws3, bs3, Wn3, pow3, w2g3)


def _edge_body(ts_hbm, tn_hbm, sidx_hbm, nidx_hbm, w2g_hbm, g_hbm,
               out_hbm, idx_s_all, idx_n_all,
               idx_sa, idx_na, idx_sb, idx_nb,
               rows_sa, rows_na, rows_sb, rows_nb, stage,
               wv, gv, zbuf, acc, sem_a, sem_b):
    c = lax.axis_index("c")
    s = lax.axis_index("s")
    wid = s * 2 + c
    lane0 = lax.iota(jnp.int32, 16) == 0

    def _zrow(r, _):
        for k in range(PACC // 16):
            zbuf[r, pl.ds(k * 16, 16)] = jnp.zeros((16,), jnp.float32)
        return 0

    lax.fori_loop(0, 40, _zrow, 0)
    nchunks = N // 40
    for j in range(16):
        chunk = s + 16 * j

        @pl.when(chunk < nchunks)
        def _():
            pltpu.sync_copy(zbuf, acc.at[pl.ds(chunk * 40, 40)])

    pltpu.sync_copy(w2g_hbm, wv)
    pltpu.sync_copy(g_hbm, gv)
    gsh = gv[...][0]
    wk = [wv[pl.ds(k * 16, 16)] for k in range(4)]
    pltpu.sync_copy(sidx_hbm.at[pl.ds(wid * EPW, EPW)], idx_s_all)
    pltpu.sync_copy(nidx_hbm.at[pl.ds(wid * EPW, EPW)], idx_n_all)
    plsc.subcore_barrier()

    def _load_idx(bi, idx_s, idx_n):
        for k in range(EB // 16):
            sl = pl.ds(bi * EB + k * 16, 16)
            idx_s[pl.ds(k * 16, 16)] = idx_s_all[sl]
            idx_n[pl.ds(k * 16, 16)] = idx_n_all[sl]

    def _start(idx_s, idx_n, rs, rn, sem):
        pltpu.async_copy(ts_hbm.at[idx_s], rs, sem)
        pltpu.async_copy(tn_hbm.at[idx_n], rn, sem)

    def _wait(idx_s, idx_n, rs, rn, sem):
        pltpu.make_async_copy(ts_hbm.at[idx_s], rs, sem).wait()
        pltpu.make_async_copy(tn_hbm.at[idx_n], rn, sem).wait()

    def _compute(rows_s, rows_n, idx_s):
        @plsc.parallel_loop(0, EB, step=1, unroll=8)
        def _edge(e):
            lwv = rows_n[e, pl.ds(TSW, 16)]
            pacc = jnp.zeros((16,), jnp.float32)
            for k in range(4):
                off = k * 16
                x = rows_s[e, pl.ds(off, 16)] + rows_n[e, pl.ds(off, 16)]
                pacc = pacc + jnp.maximum(x, 0.01 * x) * wk[k]
            targ = jnp.sum(pacc) + lwv[0] - gsh
            tv = jnp.exp(jnp.full((16,), targ, jnp.float32))
            for k in range(4):
                off = 64 + k * 16
                x = rows_s[e, pl.ds(off, 16)] + rows_n[e, pl.ds(off, 16)]
                stage[e, pl.ds(k * 16, 16)] = tv * jnp.maximum(x, 0.01 * x)
            stage[e, pl.ds(64, 16)] = jnp.where(lane0, tv, 0.0)

        pltpu.sync_copy(stage, acc.at[idx_s], add=True)

    _load_idx(0, idx_sa, idx_na)
    _start(idx_sa, idx_na, rows_sa, rows_na, sem_a)

    def _pair(i, _):
        _load_idx(2 * i + 1, idx_sb, idx_nb)
        _start(idx_sb, idx_nb, rows_sb, rows_nb, sem_b)
        _wait(idx_sa, idx_na, rows_sa, rows_na, sem_a)
        _compute(rows_sa, rows_na, idx_sa)
        _load_idx(2 * i + 2, idx_sa, idx_na)
        _start(idx_sa, idx_na, rows_sa, rows_na, sem_a)
        _wait(idx_sb, idx_nb, rows_sb, rows_nb, sem_b)
        _compute(rows_sb, rows_nb, idx_sb)
        return 0

    lax.fori_loop(0, (NBATCH - 1) // 2, _pair, 0)
    _wait(idx_sa, idx_na, rows_sa, rows_na, sem_a)
    _compute(rows_sa, rows_na, idx_sa)
    plsc.subcore_barrier()

    for j in range(16):
        chunk = s + 16 * j

        @pl.when(chunk < nchunks)
        def _():
            pltpu.sync_copy(acc.at[pl.ds(chunk * 40, 40)], zbuf)
            pltpu.sync_copy(zbuf, out_hbm.at[c, pl.ds(chunk * 40, 40)])


def _edge_pass(TS, TN, sidx, nidx, w2g64, g16):
    mesh = plsc.VectorSubcoreMesh(core_axis_name="c", subcore_axis_name="s")
    fn = functools.partial(
        pl.kernel,
        mesh=mesh,
        compiler_params=pltpu.CompilerParams(
            use_tc_tiling_on_sc=False, needs_layout_passes=False),
        out_type=jax.ShapeDtypeStruct((2, N, PACC), jnp.float32),
        scratch_types=[
            pltpu.VMEM((EPW,), jnp.int32),
            pltpu.VMEM((EPW,), jnp.int32),
            pltpu.VMEM((EB,), jnp.int32),
            pltpu.VMEM((EB,), jnp.int32),
            pltpu.VMEM((EB,), jnp.int32),
            pltpu.VMEM((EB,), jnp.int32),
            pltpu.VMEM((EB, TSW), jnp.float32),
            pltpu.VMEM((EB, TNW), jnp.float32),
            pltpu.VMEM((EB, TSW), jnp.float32),
            pltpu.VMEM((EB, TNW), jnp.float32),
            pltpu.VMEM((EB, PACC), jnp.float32),
            pltpu.VMEM((F,), jnp.float32),
            pltpu.VMEM((16,), jnp.float32),
            pltpu.VMEM((40, PACC), jnp.float32),
            pltpu.VMEM_SHARED((N, PACC), jnp.float32),
            pltpu.SemaphoreType.DMA,
            pltpu.SemaphoreType.DMA,
        ],
    )(_edge_body)
    return fn(TS, TN, sidx, nidx, w2g64, g16)


def _k3_body(p0_ref, p1_ref, p2_ref, fea_ref, w2bd_ref, b2_ref, logw_ref,
             w1g_ref, b1g_ref, w2gp_ref, b2gp_ref, powp_ref, w1m_ref,
             b1m_ref, x_ref):
    fea = _combine_val(p0_ref[0] + p0_ref[1], p1_ref[0] + p1_ref[1],
                       p2_ref[0] + p2_ref[1], fea_ref[...], w2bd_ref,
                       b2_ref)
    hg = _leaky(jnp.dot(fea, w1g_ref[...], preferred_element_type=jnp.float32)
                + b1g_ref[...])
    logit = (jnp.dot(hg, w2gp_ref[...], preferred_element_type=jnp.float32)
             + b2gp_ref[...] + logw_ref[...] * powp_ref[...])
    hm = _leaky(jnp.dot(fea, w1m_ref[...], preferred_element_type=jnp.float32)
                + b1m_ref[...])
    x_ref[:, :192] = hm
    x_ref[:, 192:ACCW] = logit[:, 0:16]


def _k3(P0, P1, P2, fea, logw, W2bd, b2cat, W1g_cat, b1g_cat, W2g_pad,
        b2g_pad, pow_pad, W1m_cat, b1m_cat):
    pspec = pl.BlockSpec((2, NB, PACC), lambda i: (0, i, 0))
    return pl.pallas_call(
        _k3_body,
        grid=(GN,),
        in_specs=[
            pspec, pspec, pspec,
            pl.BlockSpec((NB, F), lambda i: (i, 0)),
            pl.BlockSpec((192, 192), lambda i: (0, 0)),
            pl.BlockSpec((1, 192), lambda i: (0, 0)),
            pl.BlockSpec((NB, 1), lambda i: (i, 0)),
            pl.BlockSpec((F, 192), lambda i: (0, 0)),
            pl.BlockSpec((1, 192), lambda i: (0, 0)),
            pl.BlockSpec((192, 128), lambda i: (0, 0)),
            pl.BlockSpec((1, 128), lambda i: (0, 0)),
            pl.BlockSpec((1, 128), lambda i: (0, 0)),
            pl.BlockSpec((F, 192), lambda i: (0, 0)),
            pl.BlockSpec((1, 192), lambda i: (0, 0)),
        ],
        out_specs=pl.BlockSpec((NB, ACCW), lambda i: (i, 0)),
        out_shape=jax.ShapeDtypeStruct((N, ACCW), jnp.float32),
    )(P0, P1, P2, fea, W2bd, b2cat, logw, W1g_cat, b1g_cat, W2g_pad,
      b2g_pad, pow_pad, W1m_cat, b1m_cat)


def _crymax_body(x_ref, cry_ref, gm_ref):
    ci = pl.program_id(0)
    ni = pl.program_id(1)
    cids = ci * CB + lax.broadcasted_iota(jnp.int32, (NB, CB), 1)
    mask = cry_ref[...] == cids
    cur = jnp.full((8, CB), -1e30, jnp.float32)
    row = lax.broadcasted_iota(jnp.int32, (8, CB), 0)
    for h in range(3):
        lh = x_ref[:, 192 + h:193 + h]
        mh = jnp.max(jnp.where(mask, lh, -1e30), axis=0, keepdims=True)
        cur = jnp.where(row == h, mh, cur)

    @pl.when(ni == 0)
    def _():
        gm_ref[...] = cur

    @pl.when(ni > 0)
    def _():
        gm_ref[...] = jnp.maximum(gm_ref[...], cur)


def _crymax(X, cry2d):
    return pl.pallas_call(
        _crymax_body,
        grid=(GC, GN),
        in_specs=[
            pl.BlockSpec((NB, ACCW), lambda ci, ni: (ni, 0)),
            pl.BlockSpec((NB, 1), lambda ci, ni: (ni, 0)),
        ],
        out_specs=pl.BlockSpec((8, CB), lambda ci, ni: (0, ci)),
        out_shape=jax.ShapeDtypeStruct((8, GC * CB), jnp.float32),
    )(X, cry2d)


def _crysum_body(x_ref, cry_ref, gm_ref, w2bd_ref, b2_ref, out_ref,
                 num_ref, d_ref):
    ci = pl.program_id(0)
    ni = pl.program_id(1)
    cids = ci * CB + lax.broadcasted_iota(jnp.int32, (NB, CB), 1)
    mask = cry_ref[...] == cids
    nums = []
    d128 = jnp.zeros((CB, CB), jnp.float32)
    lane = lax.broadcasted_iota(jnp.int32, (CB, CB), 1)
    for h in range(3):
        lh = x_ref[:, 192 + h:193 + h]
        gmh = gm_ref[h:h + 1, :]
        t = jnp.where(mask, jnp.exp(lh - gmh), 0.0)
        hm = x_ref[:, h * 64:(h + 1) * 64]
        nums.append(lax.dot_general(t, hm, (((0,), (0,)), ((), ())),
                                    preferred_element_type=jnp.float32))
        sh = jnp.sum(t, axis=0)
        d128 = d128 + sh[:, None] * jnp.where(lane == h, 1.0, 0.0)
    cur = jnp.concatenate(nums, axis=1)

    @pl.when(ni == 0)
    def _():
        num_ref[...] = cur
        d_ref[...] = d128

    @pl.when(ni > 0)
    def _():
        num_ref[...] = num_ref[...] + cur
        d_ref[...] = d_ref[...] + d128

    @pl.when(ni == GN - 1)
    def _():
        msgs = jnp.dot(num_ref[...], w2bd_ref[...],
                       preferred_element_type=jnp.float32)
        tot = jnp.zeros((CB, F), jnp.float32)
        for h in range(3):
            denom = d_ref[:, h:h + 1]
            o = (msgs[:, h * 64:(h + 1) * 64]
                 + denom * b2_ref[0:1, h * 64:(h + 1) * 64])
            tot = tot + o / (denom + 1e-30)
        out_ref[...] = tot * (1.0 / 3.0)


def _crysum(X, cry2d, GM, W2bd, b2cat):
    return pl.pallas_call(
        _crysum_body,
        grid=(GC, GN),
        in_specs=[
            pl.BlockSpec((NB, ACCW), lambda ci, ni: (ni, 0)),
            pl.BlockSpec((NB, 1), lambda ci, ni: (ni, 0)),
            pl.BlockSpec((8, CB), lambda ci, ni: (0, ci)),
            pl.BlockSpec((192, 192), lambda ci, ni: (0, 0)),
            pl.BlockSpec((1, 192), lambda ci, ni: (0, 0)),
        ],
        out_specs=pl.BlockSpec((CB, F), lambda ci, ni: (ci, 0)),
        out_shape=jax.ShapeDtypeStruct((GC * CB, F), jnp.float32),
        scratch_shapes=[
            pltpu.VMEM((CB, 192), jnp.float32),
            pltpu.VMEM((CB, CB), jnp.float32),
        ],
    )(X, cry2d, GM, W2bd, b2cat)


def _layer_weights(heads):
    Ws_l, bs_l, Wn_l, pow_l, w2g_l, W2m, b2m = [], [], [], [], [], [], []
    for p in heads:
        (W1g, b1g), (w2g_, _) = p["gate"]
        (W1m, b1m), (W2m_, b2m_) = p["msg"]
        Ws_l.append(jnp.concatenate([W1g[:F], W1m[:F]], axis=1))
        bs_l.append(jnp.concatenate([b1g, b1m]).reshape(1, TSW))
        Wn_l.append(jnp.concatenate([W1g[F:], W1m[F:]], axis=1))
        pw = jnp.zeros((1, 128), jnp.float32).at[0, 0].set(p["pow"][0])
        pow_l.append(pw)
        w2g_l.append(w2g_[:, 0].reshape(1, F))
        W2m.append(W2m_)
        b2m.append(b2m_)
    W2bd = jax.scipy.linalg.block_diag(*W2m)
    b2cat = jnp.concatenate(b2m).reshape(1, 192)
    return (jnp.stack(Ws_l), jnp.stack(bs_l), jnp.stack(Wn_l),
            jnp.stack(pow_l), jnp.stack(w2g_l), W2bd, b2cat)


def kernel(elem_weights, elem_fea, params, self_fea_idx, nbr_fea_idx,
           cry_elem_idx):
    Wemb, bemb = params["emb"]
    W64 = jnp.pad(Wemb, ((0, 0), (0, 1)))
    b64 = jnp.pad(bemb, (0, 1)).reshape(1, F)
    sidx = self_fea_idx.astype(jnp.int32)
    nidx = nbr_fea_idx.astype(jnp.int32)

    lw = [_layer_weights(heads) for heads in params["graphs"]]

    fea, logw, *tg = _k0(elem_fea, elem_weights, W64, b64,
                         lw[0][0], lw[0][1], lw[0][2], lw[0][3], lw[0][4])
    for li in range(3):
        Ws3, bs3, Wn3, pow3, w2g3, W2bd, b2cat = lw[li]
        ts0, ts1, ts2, tn0, tn1, tn2, G = tg
        TSs, TNs = [ts0, ts1, ts2], [tn0, tn1, tn2]
        Ps = []
        for h in range(3):
            g16 = jnp.broadcast_to(G[0, h], (16,))
            Ps.append(_edge_pass(TSs[h], TNs[h], sidx, nidx,
                                 w2g3[h, 0], g16))
        if li < 2:
            nxt = lw[li + 1]
            fea, *tg = _kc(Ps[0], Ps[1], Ps[2], fea, logw, W2bd, b2cat,
                           nxt[0], nxt[1], nxt[2], nxt[3], nxt[4])
        else:
            W1g_cat = jnp.concatenate(
                [p["gate"][0][0] for p in params["cry"]], axis=1)
            b1g_cat = jnp.concatenate(
                [p["gate"][0][1] for p in params["cry"]]).reshape(1, 192)
            W2g_bd = jax.scipy.linalg.block_diag(
                *[p["gate"][1][0] for p in params["cry"]])
            W2g_pad = jnp.pad(W2g_bd, ((0, 0), (0, 125)))
            b2g_pad = jnp.zeros((1, 128), jnp.float32).at[0, :3].set(
                jnp.stack([p["gate"][1][1][0] for p in params["cry"]]))
            pow_pad = jnp.zeros((1, 128), jnp.float32).at[0, :3].set(
                jnp.stack([p["pow"][0] for p in params["cry"]]))
            W1m_cat = jnp.concatenate(
                [p["msg"][0][0] for p in params["cry"]], axis=1)
            b1m_cat = jnp.concatenate(
                [p["msg"][0][1] for p in params["cry"]]).reshape(1, 192)
            X = _k3(Ps[0], Ps[1], Ps[2], fea, logw, W2bd, b2cat,
                    W1g_cat, b1g_cat, W2g_pad, b2g_pad, pow_pad,
                    W1m_cat, b1m_cat)

    cry2d = cry_elem_idx.astype(jnp.int32).reshape(N, 1)
    GM = _crymax(X, cry2d)
    W2bd_c = jax.scipy.linalg.block_diag(
        *[p["msg"][1][0] for p in params["cry"]])
    b2cat_c = jnp.concatenate([p["msg"][1][1] for p in params["cry"]]
                              ).reshape(1, 192)
    OUT = _crysum(X, cry2d, GM, W2bd_c, b2cat_c)
    return OUT[:C]


# speedup vs baseline: 22.6392x; 1.0339x over previous
import functools

import jax
import jax.numpy as jnp
from jax import lax
from jax.experimental import pallas as pl
from jax.experimental.pallas import tpu as pltpu
from jax.experimental.pallas import tpu_sc as plsc

N = 10000
M = 320000
C = 1000
F = 64
NB = 1000
GN = N // NB
CB = 128
GC = (C + CB - 1) // CB
ACCW = 208
TSW = 128
TNW = 144
TSP = 64
TNP = 80
PACC = 80
NW = 32
EPW = M // NW
EB = 80
NBATCH = EPW // EB


def _leaky(x):
    return jnp.maximum(x, 0.01 * x)


def _pack_bf16(x):
    xb = lax.bitcast_convert_type(x.astype(jnp.bfloat16), jnp.uint16)
    parts = []
    for gi in range(4):
        lo = xb[:, gi * 32:gi * 32 + 16].astype(jnp.uint32)
        hi = xb[:, gi * 32 + 16:gi * 32 + 32].astype(jnp.uint32)
        parts.append(lo | (hi << 16))
    return lax.bitcast_convert_type(jnp.concatenate(parts, axis=1),
                                    jnp.int32)


def _emit_tables(i, fea, logw, ws_ref, bs_ref, wn_ref, pow_ref, w2g_ref,
                 ts_refs, tn_refs, g_ref, mm_ref):
    for h in range(3):
        a = (jnp.dot(fea, ws_ref[h], preferred_element_type=jnp.float32)
             + bs_ref[h])
        b = jnp.dot(fea, wn_ref[h], preferred_element_type=jnp.float32)
        ts_refs[h][...] = _pack_bf16(a)
        tn_refs[h][:, :TSP] = _pack_bf16(b)
        tn_refs[h][:, TSP:TNP] = lax.bitcast_convert_type(
            logw * pow_ref[h, 0:1, 0:16], jnp.int32)
        ag, bg = a[:, :64], b[:, :64]
        cur = jnp.concatenate(
            [jnp.min(ag, axis=0, keepdims=True),
             jnp.max(ag, axis=0, keepdims=True),
             jnp.min(bg, axis=0, keepdims=True),
             jnp.max(bg, axis=0, keepdims=True)], axis=0)
        col = pl.ds(h * 64, 64)

        @pl.when(i == 0)
        def _():
            mm_ref[0:4, col] = cur

        @pl.when(i > 0)
        def _():
            old = mm_ref[0:4, col]
            lo = jnp.minimum(old, cur)
            hi = jnp.maximum(old, cur)
            sel = lax.broadcasted_iota(jnp.int32, (4, 64), 0) % 2
            mm_ref[0:4, col] = jnp.where(sel == 0, lo, hi)

    @pl.when(i == GN - 1)
    def _():
        lane = lax.broadcasted_iota(jnp.int32, (1, 128), 1)
        g = jnp.zeros((1, 128), jnp.float32)
        for h in range(3):
            col = pl.ds(h * 64, 64)
            w = w2g_ref[h]
            t = jnp.maximum(
                w * _leaky(mm_ref[1:2, col] + mm_ref[3:4, col]),
                w * _leaky(mm_ref[0:1, col] + mm_ref[2:3, col]))
            g = g + jnp.sum(t) * jnp.where(lane == h, 1.0, 0.0)
        g_ref[...] = g


_TABLE_IN_SPECS = [
    pl.BlockSpec((3, F, TSW), lambda i: (0, 0, 0)),
    pl.BlockSpec((3, 1, TSW), lambda i: (0, 0, 0)),
    pl.BlockSpec((3, F, TSW), lambda i: (0, 0, 0)),
    pl.BlockSpec((3, 1, 128), lambda i: (0, 0, 0)),
    pl.BlockSpec((3, 1, F), lambda i: (0, 0, 0)),
]

_TABLE_OUT_SPECS = (
    [pl.BlockSpec((NB, TSP), lambda i: (i, 0)) for _ in range(3)]
    + [pl.BlockSpec((NB, TNP), lambda i: (i, 0)) for _ in range(3)]
    + [pl.BlockSpec((1, 128), lambda i: (0, 0))]
)

_TABLE_OUT_SHAPES = (
    [jax.ShapeDtypeStruct((N, TSP), jnp.int32) for _ in range(3)]
    + [jax.ShapeDtypeStruct((N, TNP), jnp.int32) for _ in range(3)]
    + [jax.ShapeDtypeStruct((1, 128), jnp.float32)]
)


def _k0_body(ef_ref, w_ref, b_ref, ew_ref, ws_ref, bs_ref, wn_ref, pow_ref,
             w2g_ref, fea_ref, logw_ref, ts0, ts1, ts2, tn0, tn1, tn2,
             g_ref, mm_ref):
    i = pl.program_id(0)
    y = jnp.dot(ef_ref[...], w_ref[...], preferred_element_type=jnp.float32)
    y = y + b_ref[...]
    ew = ew_ref[...]
    mask = lax.broadcasted_iota(jnp.int32, (NB, F), 1) == (F - 1)
    fea = y + jnp.where(mask, ew, 0.0)
    logw = jnp.log(ew)
    fea_ref[...] = fea
    logw_ref[...] = logw
    _emit_tables(i, fea, logw, ws_ref, bs_ref, wn_ref, pow_ref, w2g_ref,
                 [ts0, ts1, ts2], [tn0, tn1, tn2], g_ref, mm_ref)


def _k0(elem_fea, elem_weights, W64, b64, Ws3, bs3, Wn3, pow3, w2g3):
    return pl.pallas_call(
        _k0_body,
        grid=(GN,),
        in_specs=[
            pl.BlockSpec((NB, 200), lambda i: (i, 0)),
            pl.BlockSpec((200, F), lambda i: (0, 0)),
            pl.BlockSpec((1, F), lambda i: (0, 0)),
            pl.BlockSpec((NB, 1), lambda i: (i, 0)),
        ] + _TABLE_IN_SPECS,
        out_specs=[
            pl.BlockSpec((NB, F), lambda i: (i, 0)),
            pl.BlockSpec((NB, 1), lambda i: (i, 0)),
        ] + _TABLE_OUT_SPECS,
        out_shape=[
            jax.ShapeDtypeStruct((N, F), jnp.float32),
            jax.ShapeDtypeStruct((N, 1), jnp.float32),
        ] + _TABLE_OUT_SHAPES,
        scratch_shapes=[pltpu.VMEM((8, 192), jnp.float32)],
    )(elem_fea, W64, b64, elem_weights, Ws3, bs3, Wn3, pow3, w2g3)


def _combine_val(p0, p1, p2, fea, w2bd_ref, b2_ref):
    ps = [p0, p1, p2]
    numer = jnp.concatenate([p[:, :F] for p in ps], axis=1)
    msgs = jnp.dot(numer, w2bd_ref[...], preferred_element_type=jnp.float32)
    tot = jnp.zeros((NB, F), jnp.float32)
    for h in range(3):
        denom = ps[h][:, F:F + 1]
        o = (msgs[:, h * F:(h + 1) * F]
             + denom * b2_ref[0:1, h * F:(h + 1) * F])
        tot = tot + o / (denom + 1e-30)
    return fea + tot * (1.0 / 3.0)


def _kc_body(p0_ref, p1_ref, p2_ref, fea_ref, w2bd_ref, b2_ref, logw_ref,
             ws_ref, bs_ref, wn_ref, pow_ref, w2g_ref,
             feao_ref, ts0, ts1, ts2, tn0, tn1, tn2, g_ref, mm_ref):
    i = pl.program_id(0)
    fea = _combine_val(p0_ref[0] + p0_ref[1], p1_ref[0] + p1_ref[1],
                       p2_ref[0] + p2_ref[1], fea_ref[...], w2bd_ref,
                       b2_ref)
    feao_ref[...] = fea
    _emit_tables(i, fea, logw_ref[...], ws_ref, bs_ref, wn_ref, pow_ref,
                 w2g_ref, [ts0, ts1, ts2], [tn0, tn1, tn2], g_ref, mm_ref)


def _kc(P0, P1, P2, fea, logw, W2bd, b2cat, Ws3, bs3, Wn3, pow3, w2g3):
    pspec = pl.BlockSpec((2, NB, PACC), lambda i: (0, i, 0))
    return pl.pallas_call(
        _kc_body,
        grid=(GN,),
        in_specs=[
            pspec, pspec, pspec,
            pl.BlockSpec((NB, F), lambda i: (i, 0)),
            pl.BlockSpec((192, 192), lambda i: (0, 0)),
            pl.BlockSpec((1, 192), lambda i: (0, 0)),
            pl.BlockSpec((NB, 1), lambda i: (i, 0)),
        ] + _TABLE_IN_SPECS,
        out_specs=[pl.BlockSpec((NB, F), lambda i: (i, 0))]
        + _TABLE_OUT_SPECS,
        out_shape=[jax.ShapeDtypeStruct((N, F), jnp.float32)]
        + _TABLE_OUT_SHAPES,
        scratch_shapes=[pltpu.VMEM((8, 192), jnp.float32)],
    )(P0, P1, P2, fea, W2bd, b2cat, logw, Ws3, bs3, Wn3, pow3, w2g3)


def _edge_body(ts_hbm, tn_hbm, sidx_hbm, nidx_hbm, w2g_hbm, g_hbm,
               out_hbm, idx_s_all, idx_n_all,
               idx_sa, idx_na, idx_sb, idx_nb,
               rows_sa, rows_na, rows_sb, rows_nb, stage,
               wv, gv, zbuf, acc, sem_a, sem_b):
    c = lax.axis_index("c")
    s = lax.axis_index("s")
    wid = s * 2 + c
    lane0 = lax.iota(jnp.int32, 16) == 0

    def _zrow(r, _):
        for k in range(PACC // 16):
            zbuf[r, pl.ds(k * 16, 16)] = jnp.zeros((16,), jnp.float32)
        return 0

    lax.fori_loop(0, 40, _zrow, 0)
    nchunks = N // 40
    for j in range(16):
        chunk = s + 16 * j

        @pl.when(chunk < nchunks)
        def _():
            pltpu.sync_copy(zbuf, acc.at[pl.ds(chunk * 40, 40)])

    pltpu.sync_copy(w2g_hbm, wv)
    pltpu.sync_copy(g_hbm, gv)
    gsh = gv[...][0]
    wk = [wv[pl.ds(k * 16, 16)] for k in range(4)]
    pltpu.sync_copy(sidx_hbm.at[pl.ds(wid * EPW, EPW)], idx_s_all)
    pltpu.sync_copy(nidx_hbm.at[pl.ds(wid * EPW, EPW)], idx_n_all)
    plsc.subcore_barrier()

    def _load_idx(bi, idx_s, idx_n):
        for k in range(EB // 16):
            sl = pl.ds(bi * EB + k * 16, 16)
            idx_s[pl.ds(k * 16, 16)] = idx_s_all[sl]
            idx_n[pl.ds(k * 16, 16)] = idx_n_all[sl]

    def _start(idx_s, idx_n, rs, rn, sem):
        pltpu.async_copy(ts_hbm.at[idx_s], rs, sem)
        pltpu.async_copy(tn_hbm.at[idx_n], rn, sem)

    def _wait(idx_s, idx_n, rs, rn, sem):
        pltpu.make_async_copy(ts_hbm.at[idx_s], rs, sem).wait()
        pltpu.make_async_copy(tn_hbm.at[idx_n], rn, sem).wait()

    himask = jnp.full((16,), -65536, jnp.int32)

    def _unpk(w):
        return (plsc.bitcast(w << 16, jnp.float32),
                plsc.bitcast(w & himask, jnp.float32))

    def _compute(rows_s, rows_n, idx_s):
        @plsc.parallel_loop(0, EB, step=1, unroll=8)
        def _edge(e):
            lw = plsc.bitcast(rows_n[e, pl.ds(TSP, 16)], jnp.float32)[0]
            sg = _unpk(rows_s[e, pl.ds(0, 16)]) + _unpk(rows_s[e, pl.ds(16, 16)])
            ng = _unpk(rows_n[e, pl.ds(0, 16)]) + _unpk(rows_n[e, pl.ds(16, 16)])
            pacc = jnp.zeros((16,), jnp.float32)
            for k in range(4):
                x = sg[k] + ng[k]
                pacc = pacc + jnp.maximum(x, 0.01 * x) * wk[k]
            targ = jnp.sum(pacc) + lw - gsh
            tv = jnp.exp(jnp.full((16,), targ, jnp.float32))
            sm = _unpk(rows_s[e, pl.ds(32, 16)]) + _unpk(rows_s[e, pl.ds(48, 16)])
            nm = _unpk(rows_n[e, pl.ds(32, 16)]) + _unpk(rows_n[e, pl.ds(48, 16)])
            for k in range(4):
                x = sm[k] + nm[k]
                stage[e, pl.ds(k * 16, 16)] = tv * jnp.maximum(x, 0.01 * x)
            stage[e, pl.ds(64, 16)] = jnp.where(lane0, tv, 0.0)

        pltpu.sync_copy(stage, acc.at[idx_s], add=True)

    _load_idx(0, idx_sa, idx_na)
    _start(idx_sa, idx_na, rows_sa, rows_na, sem_a)

    def _pair(i, _):
        _load_idx(2 * i + 1, idx_sb, idx_nb)
        _start(idx_sb, idx_nb, rows_sb, rows_nb, sem_b)
        _wait(idx_sa, idx_na, rows_sa, rows_na, sem_a)
        _compute(rows_sa, rows_na, idx_sa)
        _load_idx(2 * i + 2, idx_sa, idx_na)
        _start(idx_sa, idx_na, rows_sa, rows_na, sem_a)
        _wait(idx_sb, idx_nb, rows_sb, rows_nb, sem_b)
        _compute(rows_sb, rows_nb, idx_sb)
        return 0

    lax.fori_loop(0, (NBATCH - 1) // 2, _pair, 0)
    _wait(idx_sa, idx_na, rows_sa, rows_na, sem_a)
    _compute(rows_sa, rows_na, idx_sa)
    plsc.subcore_barrier()

    for j in range(16):
        chunk = s + 16 * j

        @pl.when(chunk < nchunks)
        def _():
            pltpu.sync_copy(acc.at[pl.ds(chunk * 40, 40)], zbuf)
            pltpu.sync_copy(zbuf, out_hbm.at[c, pl.ds(chunk * 40, 40)])


def _edge_pass(TS, TN, sidx, nidx, w2g64, g16):
    mesh = plsc.VectorSubcoreMesh(core_axis_name="c", subcore_axis_name="s")
    fn = functools.partial(
        pl.kernel,
        mesh=mesh,
        compiler_params=pltpu.CompilerParams(
            use_tc_tiling_on_sc=False, needs_layout_passes=False),
        out_type=jax.ShapeDtypeStruct((2, N, PACC), jnp.float32),
        scratch_types=[
            pltpu.VMEM((EPW,), jnp.int32),
            pltpu.VMEM((EPW,), jnp.int32),
            pltpu.VMEM((EB,), jnp.int32),
            pltpu.VMEM((EB,), jnp.int32),
            pltpu.VMEM((EB,), jnp.int32),
            pltpu.VMEM((EB,), jnp.int32),
            pltpu.VMEM((EB, TSP), jnp.int32),
            pltpu.VMEM((EB, TNP), jnp.int32),
            pltpu.VMEM((EB, TSP), jnp.int32),
            pltpu.VMEM((EB, TNP), jnp.int32),
            pltpu.VMEM((EB, PACC), jnp.float32),
            pltpu.VMEM((F,), jnp.float32),
            pltpu.VMEM((16,), jnp.float32),
            pltpu.VMEM((40, PACC), jnp.float32),
            pltpu.VMEM_SHARED((N, PACC), jnp.float32),
            pltpu.SemaphoreType.DMA,
            pltpu.SemaphoreType.DMA,
        ],
    )(_edge_body)
    return fn(TS, TN, sidx, nidx, w2g64, g16)


def _k3_body(p0_ref, p1_ref, p2_ref, fea_ref, w2bd_ref, b2_ref, logw_ref,
             w1g_ref, b1g_ref, w2gp_ref, b2gp_ref, powp_ref, w1m_ref,
             b1m_ref, x_ref):
    fea = _combine_val(p0_ref[0] + p0_ref[1], p1_ref[0] + p1_ref[1],
                       p2_ref[0] + p2_ref[1], fea_ref[...], w2bd_ref,
                       b2_ref)
    hg = _leaky(jnp.dot(fea, w1g_ref[...], preferred_element_type=jnp.float32)
                + b1g_ref[...])
    logit = (jnp.dot(hg, w2gp_ref[...], preferred_element_type=jnp.float32)
             + b2gp_ref[...] + logw_ref[...] * powp_ref[...])
    hm = _leaky(jnp.dot(fea, w1m_ref[...], preferred_element_type=jnp.float32)
                + b1m_ref[...])
    x_ref[:, :192] = hm
    x_ref[:, 192:ACCW] = logit[:, 0:16]


def _k3(P0, P1, P2, fea, logw, W2bd, b2cat, W1g_cat, b1g_cat, W2g_pad,
        b2g_pad, pow_pad, W1m_cat, b1m_cat):
    pspec = pl.BlockSpec((2, NB, PACC), lambda i: (0, i, 0))
    return pl.pallas_call(
        _k3_body,
        grid=(GN,),
        in_specs=[
            pspec, pspec, pspec,
            pl.BlockSpec((NB, F), lambda i: (i, 0)),
            pl.BlockSpec((192, 192), lambda i: (0, 0)),
            pl.BlockSpec((1, 192), lambda i: (0, 0)),
            pl.BlockSpec((NB, 1), lambda i: (i, 0)),
            pl.BlockSpec((F, 192), lambda i: (0, 0)),
            pl.BlockSpec((1, 192), lambda i: (0, 0)),
            pl.BlockSpec((192, 128), lambda i: (0, 0)),
            pl.BlockSpec((1, 128), lambda i: (0, 0)),
            pl.BlockSpec((1, 128), lambda i: (0, 0)),
            pl.BlockSpec((F, 192), lambda i: (0, 0)),
            pl.BlockSpec((1, 192), lambda i: (0, 0)),
        ],
        out_specs=pl.BlockSpec((NB, ACCW), lambda i: (i, 0)),
        out_shape=jax.ShapeDtypeStruct((N, ACCW), jnp.float32),
    )(P0, P1, P2, fea, W2bd, b2cat, logw, W1g_cat, b1g_cat, W2g_pad,
      b2g_pad, pow_pad, W1m_cat, b1m_cat)


def _crymax_body(x_ref, cry_ref, gm_ref):
    ci = pl.program_id(0)
    ni = pl.program_id(1)
    cids = ci * CB + lax.broadcasted_iota(jnp.int32, (NB, CB), 1)
    mask = cry_ref[...] == cids
    cur = jnp.full((8, CB), -1e30, jnp.float32)
    row = lax.broadcasted_iota(jnp.int32, (8, CB), 0)
    for h in range(3):
        lh = x_ref[:, 192 + h:193 + h]
        mh = jnp.max(jnp.where(mask, lh, -1e30), axis=0, keepdims=True)
        cur = jnp.where(row == h, mh, cur)

    @pl.when(ni == 0)
    def _():
        gm_ref[...] = cur

    @pl.when(ni > 0)
    def _():
        gm_ref[...] = jnp.maximum(gm_ref[...], cur)


def _crymax(X, cry2d):
    return pl.pallas_call(
        _crymax_body,
        grid=(GC, GN),
        in_specs=[
            pl.BlockSpec((NB, ACCW), lambda ci, ni: (ni, 0)),
            pl.BlockSpec((NB, 1), lambda ci, ni: (ni, 0)),
        ],
        out_specs=pl.BlockSpec((8, CB), lambda ci, ni: (0, ci)),
        out_shape=jax.ShapeDtypeStruct((8, GC * CB), jnp.float32),
    )(X, cry2d)


def _crysum_body(x_ref, cry_ref, gm_ref, w2bd_ref, b2_ref, out_ref,
                 num_ref, d_ref):
    ci = pl.program_id(0)
    ni = pl.program_id(1)
    cids = ci * CB + lax.broadcasted_iota(jnp.int32, (NB, CB), 1)
    mask = cry_ref[...] == cids
    nums = []
    d128 = jnp.zeros((CB, CB), jnp.float32)
    lane = lax.broadcasted_iota(jnp.int32, (CB, CB), 1)
    for h in range(3):
        lh = x_ref[:, 192 + h:193 + h]
        gmh = gm_ref[h:h + 1, :]
        t = jnp.where(mask, jnp.exp(lh - gmh), 0.0)
        hm = x_ref[:, h * 64:(h + 1) * 64]
        nums.append(lax.dot_general(t, hm, (((0,), (0,)), ((), ())),
                                    preferred_element_type=jnp.float32))
        sh = jnp.sum(t, axis=0)
        d128 = d128 + sh[:, None] * jnp.where(lane == h, 1.0, 0.0)
    cur = jnp.concatenate(nums, axis=1)

    @pl.when(ni == 0)
    def _():
        num_ref[...] = cur
        d_ref[...] = d128

    @pl.when(ni > 0)
    def _():
        num_ref[...] = num_ref[...] + cur
        d_ref[...] = d_ref[...] + d128

    @pl.when(ni == GN - 1)
    def _():
        msgs = jnp.dot(num_ref[...], w2bd_ref[...],
                       preferred_element_type=jnp.float32)
        tot = jnp.zeros((CB, F), jnp.float32)
        for h in range(3):
            denom = d_ref[:, h:h + 1]
            o = (msgs[:, h * 64:(h + 1) * 64]
                 + denom * b2_ref[0:1, h * 64:(h + 1) * 64])
            tot = tot + o / (denom + 1e-30)
        out_ref[...] = tot * (1.0 / 3.0)


def _crysum(X, cry2d, GM, W2bd, b2cat):
    return pl.pallas_call(
        _crysum_body,
        grid=(GC, GN),
        in_specs=[
            pl.BlockSpec((NB, ACCW), lambda ci, ni: (ni, 0)),
            pl.BlockSpec((NB, 1), lambda ci, ni: (ni, 0)),
            pl.BlockSpec((8, CB), lambda ci, ni: (0, ci)),
            pl.BlockSpec((192, 192), lambda ci, ni: (0, 0)),
            pl.BlockSpec((1, 192), lambda ci, ni: (0, 0)),
        ],
        out_specs=pl.BlockSpec((CB, F), lambda ci, ni: (ci, 0)),
        out_shape=jax.ShapeDtypeStruct((GC * CB, F), jnp.float32),
        scratch_shapes=[
            pltpu.VMEM((CB, 192), jnp.float32),
            pltpu.VMEM((CB, CB), jnp.float32),
        ],
    )(X, cry2d, GM, W2bd, b2cat)


def _layer_weights(heads):
    Ws_l, bs_l, Wn_l, pow_l, w2g_l, W2m, b2m = [], [], [], [], [], [], []
    for p in heads:
        (W1g, b1g), (w2g_, _) = p["gate"]
        (W1m, b1m), (W2m_, b2m_) = p["msg"]
        Ws_l.append(jnp.concatenate([W1g[:F], W1m[:F]], axis=1))
        bs_l.append(jnp.concatenate([b1g, b1m]).reshape(1, TSW))
        Wn_l.append(jnp.concatenate([W1g[F:], W1m[F:]], axis=1))
        pw = jnp.zeros((1, 128), jnp.float32).at[0, 0].set(p["pow"][0])
        pow_l.append(pw)
        w2g_l.append(w2g_[:, 0].reshape(1, F))
        W2m.append(W2m_)
        b2m.append(b2m_)
    W2bd = jax.scipy.linalg.block_diag(*W2m)
    b2cat = jnp.concatenate(b2m).reshape(1, 192)
    return (jnp.stack(Ws_l), jnp.stack(bs_l), jnp.stack(Wn_l),
            jnp.stack(pow_l), jnp.stack(w2g_l), W2bd, b2cat)


def kernel(elem_weights, elem_fea, params, self_fea_idx, nbr_fea_idx,
           cry_elem_idx):
    Wemb, bemb = params["emb"]
    W64 = jnp.pad(Wemb, ((0, 0), (0, 1)))
    b64 = jnp.pad(bemb, (0, 1)).reshape(1, F)
    sidx = self_fea_idx.astype(jnp.int32)
    nidx = nbr_fea_idx.astype(jnp.int32)

    lw = [_layer_weights(heads) for heads in params["graphs"]]

    fea, logw, *tg = _k0(elem_fea, elem_weights, W64, b64,
                         lw[0][0], lw[0][1], lw[0][2], lw[0][3], lw[0][4])
    for li in range(3):
        Ws3, bs3, Wn3, pow3, w2g3, W2bd, b2cat = lw[li]
        ts0, ts1, ts2, tn0, tn1, tn2, G = tg
        TSs, TNs = [ts0, ts1, ts2], [tn0, tn1, tn2]
        Ps = []
        for h in range(3):
            g16 = jnp.broadcast_to(G[0, h], (16,))
            Ps.append(_edge_pass(TSs[h], TNs[h], sidx, nidx,
                                 w2g3[h, 0], g16))
        if li < 2:
            nxt = lw[li + 1]
            fea, *tg = _kc(Ps[0], Ps[1], Ps[2], fea, logw, W2bd, b2cat,
                           nxt[0], nxt[1], nxt[2], nxt[3], nxt[4])
        else:
            W1g_cat = jnp.concatenate(
                [p["gate"][0][0] for p in params["cry"]], axis=1)
            b1g_cat = jnp.concatenate(
                [p["gate"][0][1] for p in params["cry"]]).reshape(1, 192)
            W2g_bd = jax.scipy.linalg.block_diag(
                *[p["gate"][1][0] for p in params["cry"]])
            W2g_pad = jnp.pad(W2g_bd, ((0, 0), (0, 125)))
            b2g_pad = jnp.zeros((1, 128), jnp.float32).at[0, :3].set(
                jnp.stack([p["gate"][1][1][0] for p in params["cry"]]))
            pow_pad = jnp.zeros((1, 128), jnp.float32).at[0, :3].set(
                jnp.stack([p["pow"][0] for p in params["cry"]]))
            W1m_cat = jnp.concatenate(
                [p["msg"][0][0] for p in params["cry"]], axis=1)
            b1m_cat = jnp.concatenate(
                [p["msg"][0][1] for p in params["cry"]]).reshape(1, 192)
            X = _k3(Ps[0], Ps[1], Ps[2], fea, logw, W2bd, b2cat,
                    W1g_cat, b1g_cat, W2g_pad, b2g_pad, pow_pad,
                    W1m_cat, b1m_cat)

    cry2d = cry_elem_idx.astype(jnp.int32).reshape(N, 1)
    GM = _crymax(X, cry2d)
    W2bd_c = jax.scipy.linalg.block_diag(
        *[p["msg"][1][0] for p in params["cry"]])
    b2cat_c = jnp.concatenate([p["msg"][1][1] for p in params["cry"]]
                              ).reshape(1, 192)
    OUT = _crysum(X, cry2d, GM, W2bd_c, b2cat_c)
    return OUT[:C]
